# Initial kernel scaffold; baseline (speedup 1.0000x reference)
import functools

import jax
import jax.numpy as jnp
from jax import lax
from jax.experimental import pallas as pl
from jax.experimental.pallas import tpu as pltpu
from jax.experimental.pallas import tpu_sc as plsc

N = 10000
E = 160000
H = 16
C = 16
NG = 2568
F32 = jnp.float32

NC, NS, L = 2, 16, 16
CH = 256
NCH = E // CH
GRP = CH // L

OW1, OB1, OG1, OBE1, OW2, OB2, OAE = 0, 48, 64, 80, 96, 224, 232
NWC = 360

_BLK = 1000


def _ln(x, g, b):
    m = jnp.mean(x, axis=-1, keepdims=True)
    v = jnp.mean((x - m) ** 2, axis=-1, keepdims=True)
    return (x - m) / jnp.sqrt(v + 1e-5) * g + b


def _lk(x, s):
    return jnp.maximum(x, s * x)


def _tc_node_body(rx, w1, b1, g1, be1, w2, b2, g2, be2, w3, b3, as1, ad1,
                  data_o, tsrc_o, tdst_o):
    x = rx[...][:, 0:1]
    h = _lk(_ln(x * w1[...] + b1[...], g1[...], be1[...]), 0.01)
    h = _lk(_ln(jnp.dot(h, w2[...]) + b2[...], g2[...], be2[...]), 0.01)
    d = jnp.dot(h, w3[...]) + b3[...]
    s1 = jnp.dot(d, as1[...])
    d1 = jnp.dot(d, ad1[...])
    z = jnp.zeros((x.shape[0], 3), F32)
    data_o[...] = d
    tsrc_o[...] = jnp.concatenate([s1, x, z], axis=1)
    tdst_o[...] = jnp.concatenate([d1, x, z], axis=1)


def _tc_node(rx8, w1, b1, g1, be1, w2, b2, g2, be2, w3, b3, as1, ad1):
    nb = N // _BLK
    full = lambda a: pl.BlockSpec(a.shape, lambda i: (0,) * a.ndim)
    row = lambda k: pl.BlockSpec((_BLK, k), lambda i: (i, 0))
    args = (w1, b1, g1, be1, w2, b2, g2, be2, w3, b3, as1, ad1)
    return pl.pallas_call(
        _tc_node_body,
        grid=(nb,),
        in_specs=[row(8)] + [full(a) for a in args],
        out_specs=[row(8), row(20), row(20)],
        out_shape=[jax.ShapeDtypeStruct((N, 8), F32),
                   jax.ShapeDtypeStruct((N, 20), F32),
                   jax.ShapeDtypeStruct((N, 20), F32)],
    )(rx8, *args)


def _tc_mid_body(aa, ab, bd1, c1b, f1w, f1b, as2, ad2,
                 d1_o, ts_o, td_o, den_o):
    acc = aa[...] + ab[...]
    den = acc[:, 0:16]
    denr = 1.0 / (den + 1e-16)
    u = acc[:, 16:144]
    b = acc.shape[0]
    dx = jnp.concatenate(
        [jnp.broadcast_to(denr[:, i:i + 1], (b, 8)) for i in range(H)], axis=1)
    t = u * dx
    d1out = jnp.dot(t, bd1[...]) + c1b[...]
    data1 = _lk(jnp.dot(d1out, f1w[...]) + f1b[...], 0.01)
    d1_o[...] = data1
    ts_o[...] = jnp.dot(data1, as2[...])
    td_o[...] = jnp.dot(data1, ad2[...])
    den_o[...] = den


def _tc_mid(acc1a, acc1b, bd1, c1b, f1w, f1b, as2, ad2):
    nb = N // _BLK
    full = lambda a: pl.BlockSpec(a.shape, lambda i: (0,) * a.ndim)
    row = lambda k: pl.BlockSpec((_BLK, k), lambda i: (i, 0))
    args = (bd1, c1b, f1w, f1b, as2, ad2)
    return pl.pallas_call(
        _tc_mid_body,
        grid=(nb,),
        in_specs=[row(144), row(144)] + [full(a) for a in args],
        out_specs=[row(16), row(16), row(16), row(16)],
        out_shape=[jax.ShapeDtypeStruct((N, 16), F32)] * 4,
    )(acc1a, acc1b, *args)


def _tc_fin_body(aa, ab, d1, bd2, c2b, f2w, f2b, dall_o):
    pieces = []
    for h in range(H):
        a = aa[...] if h < 8 else ab[...]
        hh = h % 8
        denr = 1.0 / (a[:, hh:hh + 1] + 1e-16)
        pieces.append(a[:, 8 + hh * 16:8 + (hh + 1) * 16] * denr)
    t = jnp.concatenate(pieces, axis=1)
    d2out = jnp.dot(t, bd2[...]) + c2b[...]
    data2 = _lk(jnp.dot(d2out, f2w[...]) + f2b[...], 0.01)
    dall_o[...] = d1[...] + data2


def _tc_fin(acc2a, acc2b, data1, bd2, c2b, f2w, f2b):
    nb = N // _BLK
    full = lambda a: pl.BlockSpec(a.shape, lambda i: (0,) * a.ndim)
    row = lambda k: pl.BlockSpec((_BLK, k), lambda i: (i, 0))
    args = (bd2, c2b, f2w, f2b)
    return pl.pallas_call(
        _tc_fin_body,
        grid=(nb,),
        in_specs=[row(136), row(136), row(16)] + [full(a) for a in args],
        out_specs=[row(16)],
        out_shape=[jax.ShapeDtypeStruct((N, 16), F32)],
    )(acc2a, acc2b, data1, *args)[0]


def _tc_sel_body(dall, gm_o):
    sel = dall[0:NG, :]
    m = jnp.max(sel, axis=1, keepdims=True)
    lse = m[:, 0] + jnp.log(jnp.sum(jnp.exp(sel - m), axis=1))
    gene = lse - sel[:, 0]
    cin = jnp.mean(sel, axis=1)
    z = jnp.zeros((NG, 6), F32)
    gm_o[...] = jnp.concatenate([gene[:, None], cin[:, None], z], axis=1)


def _tc_sel(dall):
    return pl.pallas_call(
        _tc_sel_body,
        out_shape=jax.ShapeDtypeStruct((NG, 8), F32),
    )(dall)


def _tc_cell_body(cin, w1, b1, g1, be1, w2, b2, g2, be2, w3, b3, ct_o):
    c = _lk(_ln(jnp.dot(cin[...], w1[...]) + b1[...], g1[...], be1[...]), 0.01)
    c = _lk(_ln(jnp.dot(c, w2[...]) + b2[...], g2[...], be2[...]), 0.01)
    lg = jnp.dot(c, w3[...]) + b3[...]
    ex = jnp.exp(lg - jnp.max(lg, axis=-1, keepdims=True))
    ct_o[...] = ex / jnp.sum(ex, axis=-1, keepdims=True)


def _tc_cell(cin, *args):
    return pl.pallas_call(
        _tc_cell_body,
        out_shape=jax.ShapeDtypeStruct((1, 19), F32),
    )(cin, *args)


_MESH = plsc.VectorSubcoreMesh(core_axis_name="c", subcore_axis_name="s")


def _iota():
    return lax.iota(jnp.int32, L)


def _splat(v):
    return jnp.full((L,), v, jnp.int32)


def _rsqrt_sc(x):
    i = lax.bitcast_convert_type(x, jnp.int32)
    i = 0x5F3759DF - lax.shift_right_logical(i, 1)
    y = lax.bitcast_convert_type(i, F32)
    for _ in range(3):
        y = y * (1.5 - 0.5 * x * y * y)
    return y


def _zero_shared(zbuf, acc_sh, s, rows_per_sub, zrows):
    def zb(i, _):
        for j in range(zbuf.shape[1] // L):
            zbuf[i, pl.ds(j * L, L)] = jnp.zeros((L,), F32)
        return 0
    lax.fori_loop(0, zrows, zb, 0)
    for r in range(rows_per_sub // zrows):
        pltpu.sync_copy(zbuf,
                        acc_sh.at[pl.ds(s * rows_per_sub + r * zrows, zrows)])


def _sc_gat1_body(src_h, dst_h, tsrc_h, tdst_h, dtab_h, wc_h,
                  e1_h, acca_h, accb_h,
                  srcv, dstv, srows, drows, drow8, echunk, contrib, wcv, zbuf,
                  acc_sh):
    c = lax.axis_index("c")
    s = lax.axis_index("s")
    wid = s * NC + c
    _zero_shared(zbuf, acc_sh, s, 625, 125)
    pltpu.sync_copy(wc_h, wcv)
    plsc.subcore_barrier()

    def chunk_body(i, _):
        cid = wid + 32 * i

        @pl.when(cid < NCH)
        def _():
            base = cid * CH
            pltpu.sync_copy(src_h.at[pl.ds(cid * 2, 2)], srcv)
            pltpu.sync_copy(dst_h.at[pl.ds(cid * 2, 2)], dstv)
            for j in range(2):
                half = pl.ds(j * 128, 128)
                pltpu.sync_copy(tsrc_h.at[srcv.at[j]], srows.at[half])
                pltpu.sync_copy(tdst_h.at[dstv.at[j]], drows.at[half])
                pltpu.sync_copy(dtab_h.at[srcv.at[j]], drow8.at[half])

            def grp(g, _g):
                rows = g * L + _iota()
                rs = plsc.load_gather(srows, [rows, _splat(16)])
                rd = plsc.load_gather(drows, [rows, _splat(16)])
                prod = rs * rd
                hv = []
                for jj in range(16):
                    t = (prod * wcv[OW1 + jj] + rs * wcv[OW1 + 16 + jj]
                         + rd * wcv[OW1 + 32 + jj] + wcv[OB1 + jj])
                    hv.append(t)
                mean = hv[0]
                for t in hv[1:]:
                    mean = mean + t
                mean = mean * (1.0 / 16.0)
                dv = [t - mean for t in hv]
                var = dv[0] * dv[0]
                for t in dv[1:]:
                    var = var + t * t
                var = var * (1.0 / 16.0)
                r = _rsqrt_sc(var + 1e-5)
                hl = [_lk(dv[jj] * r * wcv[OG1 + jj] + wcv[OBE1 + jj], 0.01)
                      for jj in range(16)]
                sig = []
                for jj in range(8):
                    t = wcv[OB2 + jj]
                    for k in range(16):
                        t = t + hl[k] * wcv[OW2 + k * 8 + jj]
                    t = _lk(t, 0.01)
                    sig.append(1.0 / (1.0 + jnp.exp(-t)))
                dk = [plsc.load_gather(drow8, [rows, _splat(k)])
                      for k in range(8)]
                for h in range(16):
                    ew = sig[0] * wcv[OAE + h]
                    for k in range(1, 8):
                        ew = ew + sig[k] * wcv[OAE + k * 16 + h]
                    sv = plsc.load_gather(srows, [rows, _splat(h)])
                    dvv = plsc.load_gather(drows, [rows, _splat(h)])
                    ev = jnp.exp(_lk(sv + dvv + ew, 0.2))
                    plsc.store_scatter(echunk, [rows, _splat(h)], ev)
                    plsc.store_scatter(contrib, [rows, _splat(h)], ev)
                    for k in range(8):
                        plsc.store_scatter(
                            contrib, [rows, _splat(16 + h * 8 + k)],
                            ev * dk[k])
                return 0

            lax.fori_loop(0, GRP, grp, 0)
            pltpu.sync_copy(echunk, e1_h.at[pl.ds(base, CH)])
            for j in range(2):
                pltpu.sync_copy(contrib.at[pl.ds(j * 128, 128)],
                                acc_sh.at[dstv.at[j]], add=True)
        return 0

    lax.fori_loop(0, 20, chunk_body, 0)
    plsc.subcore_barrier()
    rsl = pl.ds(s * 625, 625)

    @pl.when(c == 0)
    def _():
        pltpu.sync_copy(acc_sh.at[rsl], acca_h.at[rsl])

    @pl.when(c == 1)
    def _():
        pltpu.sync_copy(acc_sh.at[rsl], accb_h.at[rsl])


def _sc_gat1(src2d, dst2d, tsrc, tdst, dtab, wc):
    return pl.kernel(
        _sc_gat1_body,
        out_type=[jax.ShapeDtypeStruct((E, 16), F32),
                  jax.ShapeDtypeStruct((N, 144), F32),
                  jax.ShapeDtypeStruct((N, 144), F32)],
        mesh=_MESH,
        scratch_types=[
            pltpu.VMEM((2, 128), jnp.int32),
            pltpu.VMEM((2, 128), jnp.int32),
            pltpu.VMEM((CH, 20), F32),
            pltpu.VMEM((CH, 20), F32),
            pltpu.VMEM((CH, 8), F32),
            pltpu.VMEM((CH, 16), F32),
            pltpu.VMEM((CH, 144), F32),
            pltpu.VMEM((NWC, 16), F32),
            pltpu.VMEM((125, 144), F32),
            pltpu.VMEM_SHARED((N, 144), F32),
        ],
    )(src2d, dst2d, tsrc, tdst, dtab, wc)


def _sc_alpha_body(dst_h, e1_h, den_h, al_h,
                   dstv, erows, denrows, achunk):
    c = lax.axis_index("c")
    s = lax.axis_index("s")
    wid = s * NC + c

    def chunk_body(i, _):
        cid = wid + 32 * i

        @pl.when(cid < NCH)
        def _():
            base = cid * CH
            pltpu.sync_copy(dst_h.at[pl.ds(cid * 2, 2)], dstv)
            pltpu.sync_copy(e1_h.at[pl.ds(base, CH)], erows)
            for j in range(2):
                pltpu.sync_copy(den_h.at[dstv.at[j]],
                                denrows.at[pl.ds(j * 128, 128)])

            def row(i2, _r):
                achunk[i2] = erows[i2] / (denrows[i2] + 1e-16)
                return 0

            lax.fori_loop(0, CH, row, 0)
            pltpu.sync_copy(achunk, al_h.at[pl.ds(base, CH)])
        return 0

    lax.fori_loop(0, 20, chunk_body, 0)


def _sc_alpha(dst2d, e1, den1):
    return pl.kernel(
        _sc_alpha_body,
        out_type=jax.ShapeDtypeStruct((E, 16), F32),
        mesh=_MESH,
        scratch_types=[
            pltpu.VMEM((2, 128), jnp.int32),
            pltpu.VMEM((CH, 16), F32),
            pltpu.VMEM((CH, 16), F32),
            pltpu.VMEM((CH, 16), F32),
        ],
    )(dst2d, e1, den1)


def _sc_gat2_body(src_h, dst_h, ts_h, td_h, d1_h,
                  acca_h, accb_h,
                  srcv, dstv, srows, drows, d1rows, contrib, zbuf,
                  acc_sh):
    c = lax.axis_index("c")
    s = lax.axis_index("s")
    _zero_shared(zbuf, acc_sh, s, 625, 125)
    plsc.subcore_barrier()
    hbase = c * 8

    def chunk_body(i, _):
        cid = s + NS * i

        @pl.when(cid < NCH)
        def _():
            pltpu.sync_copy(src_h.at[pl.ds(cid * 2, 2)], srcv)
            pltpu.sync_copy(dst_h.at[pl.ds(cid * 2, 2)], dstv)
            for j in range(2):
                half = pl.ds(j * 128, 128)
                pltpu.sync_copy(ts_h.at[srcv.at[j]], srows.at[half])
                pltpu.sync_copy(td_h.at[dstv.at[j]], drows.at[half])
                pltpu.sync_copy(d1_h.at[srcv.at[j]], d1rows.at[half])

            def grp(g, _g):
                rows = g * L + _iota()
                dk = [plsc.load_gather(d1rows, [rows, _splat(k)])
                      for k in range(16)]
                for h in range(8):
                    hh = hbase + h
                    sv = plsc.load_gather(srows, [rows, _splat(hh)])
                    dvv = plsc.load_gather(drows, [rows, _splat(hh)])
                    ev = jnp.exp(_lk(sv + dvv, 0.2))
                    plsc.store_scatter(contrib, [rows, _splat(h)], ev)
                    for k in range(16):
                        plsc.store_scatter(
                            contrib, [rows, _splat(8 + h * 16 + k)],
                            ev * dk[k])
                return 0

            lax.fori_loop(0, GRP, grp, 0)
            for j in range(2):
                pltpu.sync_copy(contrib.at[pl.ds(j * 128, 128)],
                                acc_sh.at[dstv.at[j]], add=True)
        return 0

    lax.fori_loop(0, 40, chunk_body, 0)
    plsc.subcore_barrier()
    rsl = pl.ds(s * 625, 625)

    @pl.when(c == 0)
    def _():
        pltpu.sync_copy(acc_sh.at[rsl], acca_h.at[rsl])

    @pl.when(c == 1)
    def _():
        pltpu.sync_copy(acc_sh.at[rsl], accb_h.at[rsl])


def _sc_gat2(src2d, dst2d, ts2, td2, d1tab):
    return pl.kernel(
        _sc_gat2_body,
        out_type=[jax.ShapeDtypeStruct((N, 136), F32),
                  jax.ShapeDtypeStruct((N, 136), F32)],
        mesh=_MESH,
        scratch_types=[
            pltpu.VMEM((2, 128), jnp.int32),
            pltpu.VMEM((2, 128), jnp.int32),
            pltpu.VMEM((CH, 16), F32),
            pltpu.VMEM((CH, 16), F32),
            pltpu.VMEM((CH, 16), F32),
            pltpu.VMEM((CH, 136), F32),
            pltpu.VMEM((125, 136), F32),
            pltpu.VMEM_SHARED((N, 136), F32),
        ],
    )(src2d, dst2d, ts2, td2, d1tab)


def kernel(seq_data, raw_x, edge_index, edge_tf, batch, gene_num, gene_id_vec,
           params):
    p = params
    r2 = lambda a: a.reshape(1, -1)

    w1 = p['c1_w'].reshape(8, H, C)
    as1 = jnp.einsum('khc,hc->kh', w1, p['c1_as'])
    ad1 = jnp.einsum('khc,hc->kh', w1, p['c1_ad'])
    ae1 = jnp.einsum('khc,hc->kh', p['c1_we'].reshape(8, H, C), p['c1_ae'])
    w2 = p['c2_w'].reshape(16, H, C)
    as2 = jnp.einsum('khc,hc->kh', w2, p['c2_as'])
    ad2 = jnp.einsum('khc,hc->kh', w2, p['c2_ad'])
    eye = jnp.eye(H, dtype=F32)
    bd1 = (eye[:, None, :, None] * w1.transpose(1, 0, 2)[:, :, None, :]
           ).reshape(128, 256)
    bd2 = (eye[:, None, :, None] * w2.transpose(1, 0, 2)[:, :, None, :]
           ).reshape(256, 256)
    wvec = jnp.concatenate([
        p['e_w1'].reshape(-1), p['e_b1'], p['e_g1'], p['e_be1'],
        p['e_w2'].reshape(-1), p['e_b2'], ae1.reshape(-1)])
    wc = jnp.broadcast_to(wvec[:, None], (NWC, L)) + jnp.zeros((NWC, L), F32)

    rx8 = jnp.pad(raw_x, ((0, 0), (0, 7)))
    data8, tsrc, tdst = _tc_node(
        rx8, p['n_w1'], r2(p['n_b1']), r2(p['n_g1']), r2(p['n_be1']),
        p['n_w2'], r2(p['n_b2']), r2(p['n_g2']), r2(p['n_be2']),
        p['n_w3'], r2(p['n_b3']), as1, ad1)

    src2d = edge_index[0].reshape(E // 128, 128)
    dst2d = edge_index[1].reshape(E // 128, 128)
    e1, acc1a, acc1b = _sc_gat1(src2d, dst2d, tsrc, tdst, data8, wc)

    data1, ts2, td2, den1 = _tc_mid(
        acc1a, acc1b, bd1, r2(p['c1_b']), p['f1_w'], r2(p['f1_b']), as2, ad2)

    alpha1 = _sc_alpha(dst2d, e1, den1)

    src2t = edge_tf[0].reshape(E // 128, 128)
    dst2t = edge_tf[1].reshape(E // 128, 128)
    acc2a, acc2b = _sc_gat2(src2t, dst2t, ts2, td2, data1)

    dall = _tc_fin(acc2a, acc2b, data1, bd2, r2(p['c2_b']), p['f2_w'],
                   r2(p['f2_b']))
    gm = _tc_sel(dall)
    gene_out = gm[:, 0]
    cin = gm[:, 1].reshape(1, NG)
    ct = _tc_cell(
        cin, p['ct_w1'], r2(p['ct_b1']), r2(p['ct_g1']), r2(p['ct_be1']),
        p['ct_w2'], r2(p['ct_b2']), r2(p['ct_g2']), r2(p['ct_be2']),
        p['ct_w3'], r2(p['ct_b3']))
    cell_type = ct[0]
    return gene_out, alpha1, cell_type


# speedup vs baseline: 32.9706x; 32.9706x over previous
import functools

import jax
import jax.numpy as jnp
from jax import lax
from jax.experimental import pallas as pl
from jax.experimental.pallas import tpu as pltpu
from jax.experimental.pallas import tpu_sc as plsc

N = 10000
E = 160000
H = 16
C = 16
NG = 2568
F32 = jnp.float32

NC, NS, L = 2, 16, 16
CH = 256
NCH = E // CH
GRP = CH // L
NPAD = 10112

OW1, OB1, OG1, OBE1, OW2, OB2, OAE = 0, 48, 64, 80, 96, 224, 232
NWC = 240

_BLK = 1000
_SC_PARAMS = pltpu.CompilerParams(use_tc_tiling_on_sc=False)


def _ln(x, g, b):
    m = jnp.mean(x, axis=-1, keepdims=True)
    v = jnp.mean((x - m) ** 2, axis=-1, keepdims=True)
    return (x - m) / jnp.sqrt(v + 1e-5) * g + b


def _lk(x, s):
    return jnp.maximum(x, s * x)


def _tc_node_body(rx, w1, b1, g1, be1, w2, b2, g2, be2, w3, b3, as1, ad1,
                  data_o, tsrc_o, tdst_o):
    x = rx[...][:, 0:1]
    h = _lk(_ln(x * w1[...] + b1[...], g1[...], be1[...]), 0.01)
    h = _lk(_ln(jnp.dot(h, w2[...]) + b2[...], g2[...], be2[...]), 0.01)
    d = jnp.dot(h, w3[...]) + b3[...]
    data_o[...] = jnp.concatenate([d, jnp.zeros((x.shape[0], 8), F32)], axis=1)
    tsrc_o[...] = jnp.dot(d, as1[...])
    tdst_o[...] = jnp.dot(d, ad1[...])


def _tc_node(rx8, w1, b1, g1, be1, w2, b2, g2, be2, w3, b3, as1, ad1):
    nb = N // _BLK
    full = lambda a: pl.BlockSpec(a.shape, lambda i: (0,) * a.ndim)
    row = lambda k: pl.BlockSpec((_BLK, k), lambda i: (i, 0))
    args = (w1, b1, g1, be1, w2, b2, g2, be2, w3, b3, as1, ad1)
    return pl.pallas_call(
        _tc_node_body,
        grid=(nb,),
        in_specs=[row(8)] + [full(a) for a in args],
        out_specs=[row(16), row(16), row(16)],
        out_shape=[jax.ShapeDtypeStruct((N, 16), F32),
                   jax.ShapeDtypeStruct((N, 16), F32),
                   jax.ShapeDtypeStruct((N, 16), F32)],
    )(rx8, *args)


def _tc_mid_body(a80, b80, a64, b64, bd1, c1b, f1w, f1b, as2, ad2,
                 d1_o, ts_o, td_o, den_o):
    den = a80[...][:, 0:16] + b80[...][:, 0:16]
    denr = 1.0 / (den + 1e-16)
    u = jnp.concatenate([a80[...][:, 16:80] + b80[...][:, 16:80],
                         a64[...] + b64[...]], axis=1)
    dx = jnp.concatenate([denr] * 8, axis=1)
    t = u * dx
    d1out = jnp.dot(t, bd1[...]) + c1b[...]
    data1 = _lk(jnp.dot(d1out, f1w[...]) + f1b[...], 0.01)
    d1_o[...] = data1
    ts_o[...] = jnp.dot(data1, as2[...])
    td_o[...] = jnp.dot(data1, ad2[...])
    den_o[...] = den


def _tc_mid(acc1a, acc1b, u1a, u1b, bd1, c1b, f1w, f1b, as2, ad2):
    nb = N // _BLK
    full = lambda a: pl.BlockSpec(a.shape, lambda i: (0,) * a.ndim)
    row = lambda k: pl.BlockSpec((_BLK, k), lambda i: (i, 0))
    args = (bd1, c1b, f1w, f1b, as2, ad2)
    return pl.pallas_call(
        _tc_mid_body,
        grid=(nb,),
        in_specs=[row(80), row(80), row(64), row(64)]
        + [full(a) for a in args],
        out_specs=[row(16), row(16), row(16), row(16)],
        out_shape=[jax.ShapeDtypeStruct((N, 16), F32)] * 4,
    )(acc1a, acc1b, u1a, u1b, *args)


def _tc_fin_body(a80, b80, ab1, bb1, ab2, bb2, d1, bd2, c2b, f2w, f2b,
                 dall_o):
    den = a80[...][:, 0:16] + b80[...][:, 0:16]
    denr = 1.0 / (den + 1e-16)
    u = jnp.concatenate([a80[...][:, 16:80] + b80[...][:, 16:80],
                         ab1[...] + bb1[...],
                         ab2[...] + bb2[...]], axis=1)
    dx = jnp.concatenate([denr] * 16, axis=1)
    t = u * dx
    d2out = jnp.dot(t, bd2[...]) + c2b[...]
    data2 = _lk(jnp.dot(d2out, f2w[...]) + f2b[...], 0.01)
    dall_o[...] = d1[...] + data2


def _tc_fin(a80, b80, ab1, bb1, ab2, bb2, data1, bd2, c2b, f2w, f2b):
    nb = N // _BLK
    full = lambda a: pl.BlockSpec(a.shape, lambda i: (0,) * a.ndim)
    row = lambda k: pl.BlockSpec((_BLK, k), lambda i: (i, 0))
    args = (bd2, c2b, f2w, f2b)
    return pl.pallas_call(
        _tc_fin_body,
        grid=(nb,),
        in_specs=[row(80), row(80), row(96), row(96), row(96), row(96),
                  row(16)] + [full(a) for a in args],
        out_specs=[row(16)],
        out_shape=[jax.ShapeDtypeStruct((N, 16), F32)],
    )(a80, b80, ab1, bb1, ab2, bb2, data1, *args)[0]


def _tc_sel_body(dall, gm_o):
    sel = dall[0:NG, :]
    m = jnp.max(sel, axis=1, keepdims=True)
    lse = m[:, 0] + jnp.log(jnp.sum(jnp.exp(sel - m), axis=1))
    gene = lse - sel[:, 0]
    cin = jnp.mean(sel, axis=1)
    z = jnp.zeros((NG, 6), F32)
    gm_o[...] = jnp.concatenate([gene[:, None], cin[:, None], z], axis=1)


def _tc_sel(dall):
    return pl.pallas_call(
        _tc_sel_body,
        out_shape=jax.ShapeDtypeStruct((NG, 8), F32),
    )(dall)


def _tc_cell_body(cin, w1, b1, g1, be1, w2, b2, g2, be2, w3, b3, ct_o):
    c = _lk(_ln(jnp.dot(cin[...], w1[...]) + b1[...], g1[...], be1[...]), 0.01)
    c = _lk(_ln(jnp.dot(c, w2[...]) + b2[...], g2[...], be2[...]), 0.01)
    lg = jnp.dot(c, w3[...]) + b3[...]
    ex = jnp.exp(lg - jnp.max(lg, axis=-1, keepdims=True))
    ct_o[...] = ex / jnp.sum(ex, axis=-1, keepdims=True)


def _tc_cell(cin, *args):
    return pl.pallas_call(
        _tc_cell_body,
        out_shape=jax.ShapeDtypeStruct((1, 19), F32),
    )(cin, *args)


@functools.cache
def _mesh():
    return plsc.VectorSubcoreMesh(core_axis_name="c", subcore_axis_name="s")


def _rsqrt_sc(x):
    i = lax.bitcast_convert_type(x, jnp.int32)
    i = 0x5F3759DF - lax.shift_right_logical(i, 1)
    y = lax.bitcast_convert_type(i, F32)
    for _ in range(3):
        y = y * (1.5 - 0.5 * x * y * y)
    return y


def _zero_shared(zbuf, acc_sh, s):
    def zb(i, _):
        for j in range(zbuf.shape[1] // L):
            zbuf[i, pl.ds(j * L, L)] = jnp.zeros((L,), F32)
        return 0
    lax.fori_loop(0, 128, zb, 0)
    for r in range(4):
        pltpu.sync_copy(zbuf, acc_sh.at[pl.ds(s * 632 + r * 128, 128)])
    pltpu.sync_copy(zbuf.at[pl.ds(0, 120)], acc_sh.at[pl.ds(s * 632 + 512, 120)])


def _writeout(acc_sh, out_h, s):
    @pl.when(s < 15)
    def _():
        pltpu.sync_copy(acc_sh.at[pl.ds(s * 632, 632)],
                        out_h.at[pl.ds(s * 632, 632)])

    @pl.when(s == 15)
    def _():
        pltpu.sync_copy(acc_sh.at[pl.ds(9480, 520)],
                        out_h.at[pl.ds(9480, 520)])


def _edge_mlp_group(rs, rd, wcv):
    prod = rs * rd
    hv = []
    for jj in range(16):
        t = (prod * wcv[OW1 + jj, pl.ds(0, L)]
             + rs * wcv[OW1 + 16 + jj, pl.ds(0, L)]
             + rd * wcv[OW1 + 32 + jj, pl.ds(0, L)]
             + wcv[OB1 + jj, pl.ds(0, L)])
        hv.append(t)
    mean = hv[0]
    for t in hv[1:]:
        mean = mean + t
    mean = mean * (1.0 / 16.0)
    dv = [t - mean for t in hv]
    var = dv[0] * dv[0]
    for t in dv[1:]:
        var = var + t * t
    var = var * (1.0 / 16.0)
    r = _rsqrt_sc(var + 1e-5)
    hl = [_lk(dv[jj] * r * wcv[OG1 + jj, pl.ds(0, L)]
              + wcv[OBE1 + jj, pl.ds(0, L)], 0.01)
          for jj in range(16)]
    sig = []
    for jj in range(8):
        t = wcv[OB2 + jj, pl.ds(0, L)]
        for k in range(16):
            t = t + hl[k] * wcv[OW2 + k * 8 + jj, pl.ds(0, L)]
        t = _lk(t, 0.01)
        sig.append(1.0 / (1.0 + jnp.exp(-t)))
    return sig


def _sc_gat1_body(src_h, dst_h, tsrc_h, tdst_h, dtab_h, rx_h, wc_h,
                  e1_h, acca_h, accb_h,
                  sva, svb, dva, dvb, srows, drows, drow16, rxs, rxd,
                  echunk, contrib, wcv, zbuf, acc_sh):
    c = lax.axis_index("c")
    s = lax.axis_index("s")
    wid = s * NC + c
    _zero_shared(zbuf, acc_sh, s)
    pltpu.sync_copy(wc_h, wcv)
    plsc.subcore_barrier()

    def chunk_body(i, _):
        cid = wid + 32 * i

        @pl.when(cid < NCH)
        def _():
            base = cid * CH
            pltpu.sync_copy(src_h.at[pl.ds(base, 128)], sva)
            pltpu.sync_copy(src_h.at[pl.ds(base + 128, 128)], svb)
            pltpu.sync_copy(dst_h.at[pl.ds(base, 128)], dva)
            pltpu.sync_copy(dst_h.at[pl.ds(base + 128, 128)], dvb)
            for j, (sv_, dv_) in enumerate(((sva, dva), (svb, dvb))):
                half = pl.ds(j * 128, 128)
                pltpu.sync_copy(tsrc_h.at[sv_], srows.at[half])
                pltpu.sync_copy(tdst_h.at[dv_], drows.at[half])
                pltpu.sync_copy(dtab_h.at[sv_], drow16.at[half])
                pltpu.sync_copy(rx_h.at[sv_], rxs.at[half])
                pltpu.sync_copy(rx_h.at[dv_], rxd.at[half])

            def grp(g, _g):
                rs = rxs[pl.ds(g * L, L)]
                rd = rxd[pl.ds(g * L, L)]
                sig = _edge_mlp_group(rs, rd, wcv)
                for e in range(L):
                    i2 = g * L + e
                    sv = srows[i2, pl.ds(0, L)]
                    dvv = drows[i2, pl.ds(0, L)]
                    ew = sig[0][e] * wcv[OAE + 0, pl.ds(0, L)]
                    for k in range(1, 8):
                        ew = ew + sig[k][e] * wcv[OAE + k, pl.ds(0, L)]
                    ev = jnp.exp(_lk(sv + dvv + ew, 0.2))
                    echunk[i2, pl.ds(0, L)] = ev
                    contrib[i2, pl.ds(0, L)] = ev
                    dvec = drow16[i2, pl.ds(0, L)]
                    for k in range(4):
                        contrib[i2, pl.ds(16 + k * 16, L)] = ev * dvec[k]
                return 0

            lax.fori_loop(0, GRP, grp, 0)
            pltpu.sync_copy(echunk, e1_h.at[pl.ds(base, CH)])
            pltpu.sync_copy(contrib.at[pl.ds(0, 128)], acc_sh.at[dva],
                            add=True)
            pltpu.sync_copy(contrib.at[pl.ds(128, 128)], acc_sh.at[dvb],
                            add=True)
        return 0

    lax.fori_loop(0, 20, chunk_body, 0)
    plsc.subcore_barrier()

    @pl.when(c == 0)
    def _():
        _writeout(acc_sh, acca_h, s)

    @pl.when(c == 1)
    def _():
        _writeout(acc_sh, accb_h, s)


def _sc_gat1(src1, dst1, tsrc, tdst, dtab, rx1, wc):
    return pl.kernel(
        _sc_gat1_body,
        out_type=[jax.ShapeDtypeStruct((E, 16), F32),
                  jax.ShapeDtypeStruct((N, 80), F32),
                  jax.ShapeDtypeStruct((N, 80), F32)],
        mesh=_mesh(),
        compiler_params=_SC_PARAMS,
        scratch_types=[
            pltpu.VMEM((128,), jnp.int32),
            pltpu.VMEM((128,), jnp.int32),
            pltpu.VMEM((128,), jnp.int32),
            pltpu.VMEM((128,), jnp.int32),
            pltpu.VMEM((CH, 16), F32),
            pltpu.VMEM((CH, 16), F32),
            pltpu.VMEM((CH, 16), F32),
            pltpu.VMEM((CH,), F32),
            pltpu.VMEM((CH,), F32),
            pltpu.VMEM((CH, 16), F32),
            pltpu.VMEM((CH, 80), F32),
            pltpu.VMEM((NWC, 16), F32),
            pltpu.VMEM((128, 80), F32),
            pltpu.VMEM_SHARED((NPAD, 80), F32),
        ],
    )(src1, dst1, tsrc, tdst, dtab, rx1, wc)


def _sc_alpha_body(dst_h, e1_h, den_h, al_h,
                   dva, dvb, erows, denrows, achunk):
    c = lax.axis_index("c")
    s = lax.axis_index("s")
    wid = s * NC + c

    def chunk_body(i, _):
        cid = wid + 32 * i

        @pl.when(cid < NCH)
        def _():
            base = cid * CH
            pltpu.sync_copy(dst_h.at[pl.ds(base, 128)], dva)
            pltpu.sync_copy(dst_h.at[pl.ds(base + 128, 128)], dvb)
            pltpu.sync_copy(e1_h.at[pl.ds(base, CH)], erows)
            pltpu.sync_copy(den_h.at[dva], denrows.at[pl.ds(0, 128)])
            pltpu.sync_copy(den_h.at[dvb], denrows.at[pl.ds(128, 128)])

            def row(i2, _r):
                ev = erows[i2, pl.ds(0, L)]
                dn = denrows[i2, pl.ds(0, L)]
                achunk[i2, pl.ds(0, L)] = ev / (dn + 1e-16)
                return 0

            lax.fori_loop(0, CH, row, 0)
            pltpu.sync_copy(achunk, al_h.at[pl.ds(base, CH)])
        return 0

    lax.fori_loop(0, 20, chunk_body, 0)


def _sc_alpha(dst1, e1, den1):
    return pl.kernel(
        _sc_alpha_body,
        out_type=jax.ShapeDtypeStruct((E, 16), F32),
        mesh=_mesh(),
        compiler_params=_SC_PARAMS,
        scratch_types=[
            pltpu.VMEM((128,), jnp.int32),
            pltpu.VMEM((128,), jnp.int32),
            pltpu.VMEM((CH, 16), F32),
            pltpu.VMEM((CH, 16), F32),
            pltpu.VMEM((CH, 16), F32),
        ],
    )(dst1, e1, den1)


def _sc_gat2_body(src_h, dst_h, ts_h, td_h, d1_h,
                  e2_h, acca_h, accb_h,
                  sva, svb, dva, dvb, srows, drows, d1rows, echunk, contrib,
                  zbuf, acc_sh):
    c = lax.axis_index("c")
    s = lax.axis_index("s")
    wid = s * NC + c
    _zero_shared(zbuf, acc_sh, s)
    plsc.subcore_barrier()

    def chunk_body(i, _):
        cid = wid + 32 * i

        @pl.when(cid < NCH)
        def _():
            base = cid * CH
            pltpu.sync_copy(src_h.at[pl.ds(base, 128)], sva)
            pltpu.sync_copy(src_h.at[pl.ds(base + 128, 128)], svb)
            pltpu.sync_copy(dst_h.at[pl.ds(base, 128)], dva)
            pltpu.sync_copy(dst_h.at[pl.ds(base + 128, 128)], dvb)
            for j, (sv_, dv_) in enumerate(((sva, dva), (svb, dvb))):
                half = pl.ds(j * 128, 128)
                pltpu.sync_copy(ts_h.at[sv_], srows.at[half])
                pltpu.sync_copy(td_h.at[dv_], drows.at[half])
                pltpu.sync_copy(d1_h.at[sv_], d1rows.at[half])

            def grp(g, _g):
                for e in range(L):
                    i2 = g * L + e
                    sv = srows[i2, pl.ds(0, L)]
                    dvv = drows[i2, pl.ds(0, L)]
                    ev = jnp.exp(_lk(sv + dvv, 0.2))
                    echunk[i2, pl.ds(0, L)] = ev
                    contrib[i2, pl.ds(0, L)] = ev
                    dvec = d1rows[i2, pl.ds(0, L)]
                    for k in range(4):
                        contrib[i2, pl.ds(16 + k * 16, L)] = ev * dvec[k]
                return 0

            lax.fori_loop(0, GRP, grp, 0)
            pltpu.sync_copy(echunk, e2_h.at[pl.ds(base, CH)])
            pltpu.sync_copy(contrib.at[pl.ds(0, 128)], acc_sh.at[dva],
                            add=True)
            pltpu.sync_copy(contrib.at[pl.ds(128, 128)], acc_sh.at[dvb],
                            add=True)
        return 0

    lax.fori_loop(0, 20, chunk_body, 0)
    plsc.subcore_barrier()

    @pl.when(c == 0)
    def _():
        _writeout(acc_sh, acca_h, s)

    @pl.when(c == 1)
    def _():
        _writeout(acc_sh, accb_h, s)


def _sc_gat2(src1, dst1, ts2, td2, d1tab):
    return pl.kernel(
        _sc_gat2_body,
        out_type=[jax.ShapeDtypeStruct((E, 16), F32),
                  jax.ShapeDtypeStruct((N, 80), F32),
                  jax.ShapeDtypeStruct((N, 80), F32)],
        mesh=_mesh(),
        compiler_params=_SC_PARAMS,
        scratch_types=[
            pltpu.VMEM((128,), jnp.int32),
            pltpu.VMEM((128,), jnp.int32),
            pltpu.VMEM((128,), jnp.int32),
            pltpu.VMEM((128,), jnp.int32),
            pltpu.VMEM((CH, 16), F32),
            pltpu.VMEM((CH, 16), F32),
            pltpu.VMEM((CH, 16), F32),
            pltpu.VMEM((CH, 16), F32),
            pltpu.VMEM((CH, 80), F32),
            pltpu.VMEM((128, 80), F32),
            pltpu.VMEM_SHARED((NPAD, 80), F32),
        ],
    )(src1, dst1, ts2, td2, d1tab)


def _make_upass_body(kn, koff):
    w = 16 * kn

    def body(src_h, dst_h, e_h, d_h, acca_h, accb_h,
             sva, svb, dva, dvb, erows, drow16, contrib, zbuf, acc_sh):
        c = lax.axis_index("c")
        s = lax.axis_index("s")
        wid = s * NC + c
        _zero_shared(zbuf, acc_sh, s)
        plsc.subcore_barrier()

        def chunk_body(i, _):
            cid = wid + 32 * i

            @pl.when(cid < NCH)
            def _():
                base = cid * CH
                pltpu.sync_copy(src_h.at[pl.ds(base, 128)], sva)
                pltpu.sync_copy(src_h.at[pl.ds(base + 128, 128)], svb)
                pltpu.sync_copy(dst_h.at[pl.ds(base, 128)], dva)
                pltpu.sync_copy(dst_h.at[pl.ds(base + 128, 128)], dvb)
                pltpu.sync_copy(e_h.at[pl.ds(base, CH)], erows)
                pltpu.sync_copy(d_h.at[sva], drow16.at[pl.ds(0, 128)])
                pltpu.sync_copy(d_h.at[svb], drow16.at[pl.ds(128, 128)])

                def grp(g, _g):
                    for e in range(L):
                        i2 = g * L + e
                        ev = erows[i2, pl.ds(0, L)]
                        dvec = drow16[i2, pl.ds(0, L)]
                        for k in range(kn):
                            contrib[i2, pl.ds(k * 16, L)] = ev * dvec[koff + k]
                    return 0

                lax.fori_loop(0, GRP, grp, 0)
                pltpu.sync_copy(contrib.at[pl.ds(0, 128)], acc_sh.at[dva],
                                add=True)
                pltpu.sync_copy(contrib.at[pl.ds(128, 128)], acc_sh.at[dvb],
                                add=True)
            return 0

        lax.fori_loop(0, 20, chunk_body, 0)
        plsc.subcore_barrier()

        @pl.when(c == 0)
        def _():
            _writeout(acc_sh, acca_h, s)

        @pl.when(c == 1)
        def _():
            _writeout(acc_sh, accb_h, s)

    return body


def _sc_upass(src1, dst1, etab, dtab, kn, koff):
    w = 16 * kn
    return pl.kernel(
        _make_upass_body(kn, koff),
        out_type=[jax.ShapeDtypeStruct((N, w), F32),
                  jax.ShapeDtypeStruct((N, w), F32)],
        mesh=_mesh(),
        compiler_params=_SC_PARAMS,
        scratch_types=[
            pltpu.VMEM((128,), jnp.int32),
            pltpu.VMEM((128,), jnp.int32),
            pltpu.VMEM((128,), jnp.int32),
            pltpu.VMEM((128,), jnp.int32),
            pltpu.VMEM((CH, 16), F32),
            pltpu.VMEM((CH, 16), F32),
            pltpu.VMEM((CH, w), F32),
            pltpu.VMEM((128, w), F32),
            pltpu.VMEM_SHARED((NPAD, w), F32),
        ],
    )(src1, dst1, etab, dtab)


def kernel(seq_data, raw_x, edge_index, edge_tf, batch, gene_num, gene_id_vec,
           params):
    p = params
    r2 = lambda a: a.reshape(1, -1)

    w1 = p['c1_w'].reshape(8, H, C)
    as1 = jnp.einsum('khc,hc->kh', w1, p['c1_as'])
    ad1 = jnp.einsum('khc,hc->kh', w1, p['c1_ad'])
    ae1 = jnp.einsum('khc,hc->kh', p['c1_we'].reshape(8, H, C), p['c1_ae'])
    w2 = p['c2_w'].reshape(16, H, C)
    as2 = jnp.einsum('khc,hc->kh', w2, p['c2_as'])
    ad2 = jnp.einsum('khc,hc->kh', w2, p['c2_ad'])
    eye = jnp.eye(H, dtype=F32)
    bd1 = (w1[:, :, None, :] * eye[None, :, :, None]).reshape(128, 256)
    bd2 = (w2[:, :, None, :] * eye[None, :, :, None]).reshape(256, 256)
    wvec = jnp.concatenate([
        p['e_w1'].reshape(-1), p['e_b1'], p['e_g1'], p['e_be1'],
        p['e_w2'].reshape(-1), p['e_b2']])
    wc = jnp.concatenate(
        [jnp.broadcast_to(wvec[:, None], (OAE, L)), ae1], axis=0)

    rx8 = jnp.pad(raw_x, ((0, 0), (0, 7)))
    rx1 = raw_x[:, 0]
    data16, tsrc, tdst = _tc_node(
        rx8, p['n_w1'], r2(p['n_b1']), r2(p['n_g1']), r2(p['n_be1']),
        p['n_w2'], r2(p['n_b2']), r2(p['n_g2']), r2(p['n_be2']),
        p['n_w3'], r2(p['n_b3']), as1, ad1)

    e1, acc1a, acc1b = _sc_gat1(edge_index[0], edge_index[1], tsrc, tdst,
                                data16, rx1, wc)
    u1a, u1b = _sc_upass(edge_index[0], edge_index[1], e1, data16, 4, 4)

    data1, ts2, td2, den1 = _tc_mid(
        acc1a, acc1b, u1a, u1b, bd1, r2(p['c1_b']), p['f1_w'], r2(p['f1_b']),
        as2, ad2)

    alpha1 = _sc_alpha(edge_index[1], e1, den1)

    e2, acc2a, acc2b = _sc_gat2(edge_tf[0], edge_tf[1], ts2, td2, data1)
    u2a, u2b = _sc_upass(edge_tf[0], edge_tf[1], e2, data1, 6, 4)
    u2c, u2d = _sc_upass(edge_tf[0], edge_tf[1], e2, data1, 6, 10)

    dall = _tc_fin(acc2a, acc2b, u2a, u2b, u2c, u2d, data1, bd2,
                   r2(p['c2_b']), p['f2_w'], r2(p['f2_b']))
    gm = _tc_sel(dall)
    gene_out = gm[:, 0]
    cin = gm[:, 1].reshape(1, NG)
    ct = _tc_cell(
        cin, p['ct_w1'], r2(p['ct_b1']), r2(p['ct_g1']), r2(p['ct_be1']),
        p['ct_w2'], r2(p['ct_b2']), r2(p['ct_g2']), r2(p['ct_be2']),
        p['ct_w3'], r2(p['ct_b3']))
    cell_type = ct[0]
    return gene_out, alpha1, cell_type


# speedup vs baseline: 33.0441x; 1.0022x over previous
import functools

import jax
import jax.numpy as jnp
from jax import lax
from jax.experimental import pallas as pl
from jax.experimental.pallas import tpu as pltpu
from jax.experimental.pallas import tpu_sc as plsc

N = 10000
E = 160000
H = 16
C = 16
NG = 2568
F32 = jnp.float32

NC, NS, L = 2, 16, 16
CH = 256
NCH = E // CH
GRP = CH // L
NPAD = 10240

OW1, OB1, OG1, OBE1, OW2, OB2, OAE = 0, 48, 64, 80, 96, 224, 232
NWC = 240

_BLK = 1000
_SC_PARAMS = pltpu.CompilerParams(use_tc_tiling_on_sc=False)


def _ln(x, g, b):
    m = jnp.mean(x, axis=-1, keepdims=True)
    v = jnp.mean((x - m) ** 2, axis=-1, keepdims=True)
    return (x - m) / jnp.sqrt(v + 1e-5) * g + b


def _lk(x, s):
    return jnp.maximum(x, s * x)


def _tc_node_body(rx, w1, b1, g1, be1, w2, b2, g2, be2, w3, b3, as1, ad1,
                  data_o, tsrc_o, tdst_o):
    x = rx[...][:, 0:1]
    h = _lk(_ln(x * w1[...] + b1[...], g1[...], be1[...]), 0.01)
    h = _lk(_ln(jnp.dot(h, w2[...]) + b2[...], g2[...], be2[...]), 0.01)
    d = jnp.dot(h, w3[...]) + b3[...]
    data_o[...] = jnp.concatenate([d, jnp.zeros((x.shape[0], 8), F32)], axis=1)
    tsrc_o[...] = jnp.dot(d, as1[...])
    tdst_o[...] = jnp.dot(d, ad1[...])


def _tc_node(rx8, w1, b1, g1, be1, w2, b2, g2, be2, w3, b3, as1, ad1):
    nb = N // _BLK
    full = lambda a: pl.BlockSpec(a.shape, lambda i: (0,) * a.ndim)
    row = lambda k: pl.BlockSpec((_BLK, k), lambda i: (i, 0))
    args = (w1, b1, g1, be1, w2, b2, g2, be2, w3, b3, as1, ad1)
    return pl.pallas_call(
        _tc_node_body,
        grid=(nb,),
        in_specs=[row(8)] + [full(a) for a in args],
        out_specs=[row(16), row(16), row(16)],
        out_shape=[jax.ShapeDtypeStruct((N, 16), F32),
                   jax.ShapeDtypeStruct((N, 16), F32),
                   jax.ShapeDtypeStruct((N, 16), F32)],
    )(rx8, *args)


def _tc_mid_body(a80, b80, a64, b64, bd1, c1b, f1w, f1b, as2, ad2,
                 d1_o, ts_o, td_o, den_o):
    den = a80[...][:, 0:16] + b80[...][:, 0:16]
    denr = 1.0 / (den + 1e-16)
    u = jnp.concatenate([a80[...][:, 16:80] + b80[...][:, 16:80],
                         a64[...] + b64[...]], axis=1)
    dx = jnp.concatenate([denr] * 8, axis=1)
    t = u * dx
    d1out = jnp.dot(t, bd1[...]) + c1b[...]
    data1 = _lk(jnp.dot(d1out, f1w[...]) + f1b[...], 0.01)
    d1_o[...] = data1
    ts_o[...] = jnp.dot(data1, as2[...])
    td_o[...] = jnp.dot(data1, ad2[...])
    den_o[...] = den


def _tc_mid(acc1a, acc1b, u1a, u1b, bd1, c1b, f1w, f1b, as2, ad2):
    nb = N // _BLK
    full = lambda a: pl.BlockSpec(a.shape, lambda i: (0,) * a.ndim)
    row = lambda k: pl.BlockSpec((_BLK, k), lambda i: (i, 0))
    args = (bd1, c1b, f1w, f1b, as2, ad2)
    return pl.pallas_call(
        _tc_mid_body,
        grid=(nb,),
        in_specs=[row(80), row(80), row(64), row(64)]
        + [full(a) for a in args],
        out_specs=[row(16), row(16), row(16), row(16)],
        out_shape=[jax.ShapeDtypeStruct((N, 16), F32)] * 4,
    )(acc1a, acc1b, u1a, u1b, *args)


def _tc_fin_body(a80, b80, ab1, bb1, ab2, bb2, d1, bd2, c2b, f2w, f2b,
                 dall_o):
    den = a80[...][:, 0:16] + b80[...][:, 0:16]
    denr = 1.0 / (den + 1e-16)
    u = jnp.concatenate([a80[...][:, 16:80] + b80[...][:, 16:80],
                         ab1[...] + bb1[...],
                         ab2[...] + bb2[...]], axis=1)
    dx = jnp.concatenate([denr] * 16, axis=1)
    t = u * dx
    d2out = jnp.dot(t, bd2[...]) + c2b[...]
    data2 = _lk(jnp.dot(d2out, f2w[...]) + f2b[...], 0.01)
    dall_o[...] = d1[...] + data2


def _tc_fin(a80, b80, ab1, bb1, ab2, bb2, data1, bd2, c2b, f2w, f2b):
    nb = N // _BLK
    full = lambda a: pl.BlockSpec(a.shape, lambda i: (0,) * a.ndim)
    row = lambda k: pl.BlockSpec((_BLK, k), lambda i: (i, 0))
    args = (bd2, c2b, f2w, f2b)
    return pl.pallas_call(
        _tc_fin_body,
        grid=(nb,),
        in_specs=[row(80), row(80), row(96), row(96), row(96), row(96),
                  row(16)] + [full(a) for a in args],
        out_specs=[row(16)],
        out_shape=[jax.ShapeDtypeStruct((N, 16), F32)],
    )(a80, b80, ab1, bb1, ab2, bb2, data1, *args)[0]


def _tc_sel_body(dall, gm_o):
    sel = dall[0:NG, :]
    m = jnp.max(sel, axis=1, keepdims=True)
    lse = m[:, 0] + jnp.log(jnp.sum(jnp.exp(sel - m), axis=1))
    gene = lse - sel[:, 0]
    cin = jnp.mean(sel, axis=1)
    z = jnp.zeros((NG, 6), F32)
    gm_o[...] = jnp.concatenate([gene[:, None], cin[:, None], z], axis=1)


def _tc_sel(dall):
    return pl.pallas_call(
        _tc_sel_body,
        out_shape=jax.ShapeDtypeStruct((NG, 8), F32),
    )(dall)


def _tc_cell_body(cin, w1, b1, g1, be1, w2, b2, g2, be2, w3, b3, ct_o):
    c = _lk(_ln(jnp.dot(cin[...], w1[...]) + b1[...], g1[...], be1[...]), 0.01)
    c = _lk(_ln(jnp.dot(c, w2[...]) + b2[...], g2[...], be2[...]), 0.01)
    lg = jnp.dot(c, w3[...]) + b3[...]
    ex = jnp.exp(lg - jnp.max(lg, axis=-1, keepdims=True))
    ct_o[...] = ex / jnp.sum(ex, axis=-1, keepdims=True)


def _tc_cell(cin, *args):
    return pl.pallas_call(
        _tc_cell_body,
        out_shape=jax.ShapeDtypeStruct((1, 19), F32),
    )(cin, *args)


@functools.cache
def _mesh():
    return plsc.VectorSubcoreMesh(core_axis_name="c", subcore_axis_name="s")


def _rsqrt_sc(x):
    i = lax.bitcast_convert_type(x, jnp.int32)
    i = 0x5F3759DF - lax.shift_right_logical(i, 1)
    y = lax.bitcast_convert_type(i, F32)
    for _ in range(3):
        y = y * (1.5 - 0.5 * x * y * y)
    return y


def _zero_shared(zbuf, acc_sh, s):
    def zb(i, _):
        for j in range(zbuf.shape[1] // L):
            zbuf[i, pl.ds(j * L, L)] = jnp.zeros((L,), F32)
        return 0
    lax.fori_loop(0, 128, zb, 0)

    def zc(r, _):
        pltpu.sync_copy(zbuf, acc_sh.at[pl.ds(s * 640 + r * 128, 128)])
        return 0
    lax.fori_loop(0, 5, zc, 0)


def _writeout(acc_sh, out_h, s):
    def wc_(r, _):
        off = s * 640 + r * 128

        @pl.when(off + 128 <= N)
        def _():
            pltpu.sync_copy(acc_sh.at[pl.ds(off, 128)],
                            out_h.at[pl.ds(off, 128)])
        return 0
    lax.fori_loop(0, 5, wc_, 0)

    @pl.when(s == 15)
    def _():
        pltpu.sync_copy(acc_sh.at[pl.ds(9984, 16)], out_h.at[pl.ds(9984, 16)])


def _edge_mlp_group(rs, rd, wcv):
    prod = rs * rd
    hv = []
    for jj in range(16):
        t = (prod * wcv[OW1 + jj, pl.ds(0, L)]
             + rs * wcv[OW1 + 16 + jj, pl.ds(0, L)]
             + rd * wcv[OW1 + 32 + jj, pl.ds(0, L)]
             + wcv[OB1 + jj, pl.ds(0, L)])
        hv.append(t)
    mean = hv[0]
    for t in hv[1:]:
        mean = mean + t
    mean = mean * (1.0 / 16.0)
    dv = [t - mean for t in hv]
    var = dv[0] * dv[0]
    for t in dv[1:]:
        var = var + t * t
    var = var * (1.0 / 16.0)
    r = _rsqrt_sc(var + 1e-5)
    hl = [_lk(dv[jj] * r * wcv[OG1 + jj, pl.ds(0, L)]
              + wcv[OBE1 + jj, pl.ds(0, L)], 0.01)
          for jj in range(16)]
    sig = []
    for jj in range(8):
        t = wcv[OB2 + jj, pl.ds(0, L)]
        for k in range(16):
            t = t + hl[k] * wcv[OW2 + k * 8 + jj, pl.ds(0, L)]
        t = _lk(t, 0.01)
        sig.append(1.0 / (1.0 + jnp.exp(-t)))
    return sig


def _sc_gat1_body(src_h, dst_h, tsrc_h, tdst_h, dtab_h, rx_h, wc_h,
                  e1_h, acca_h, accb_h,
                  sva, svb, dva, dvb, srows, drows, drow16, rxs, rxd,
                  echunk, contrib, wcv, zbuf, acc_sh):
    c = lax.axis_index("c")
    s = lax.axis_index("s")
    wid = s * NC + c
    _zero_shared(zbuf, acc_sh, s)
    pltpu.sync_copy(wc_h, wcv)
    plsc.subcore_barrier()

    def chunk_body(i, _):
        cid = wid + 32 * i

        @pl.when(cid < NCH)
        def _():
            base = cid * CH
            pltpu.sync_copy(src_h.at[pl.ds(base, 128)], sva)
            pltpu.sync_copy(src_h.at[pl.ds(base + 128, 128)], svb)
            pltpu.sync_copy(dst_h.at[pl.ds(base, 128)], dva)
            pltpu.sync_copy(dst_h.at[pl.ds(base + 128, 128)], dvb)
            for j, (sv_, dv_) in enumerate(((sva, dva), (svb, dvb))):
                half = pl.ds(j * 128, 128)
                pltpu.sync_copy(tsrc_h.at[sv_], srows.at[half])
                pltpu.sync_copy(tdst_h.at[dv_], drows.at[half])
                pltpu.sync_copy(dtab_h.at[sv_], drow16.at[half])
                pltpu.sync_copy(rx_h.at[sv_], rxs.at[half])
                pltpu.sync_copy(rx_h.at[dv_], rxd.at[half])

            def grp(g, _g):
                rs = rxs[pl.ds(g * L, L)]
                rd = rxd[pl.ds(g * L, L)]
                sig = _edge_mlp_group(rs, rd, wcv)
                for e in range(L):
                    i2 = g * L + e
                    sv = srows[i2, pl.ds(0, L)]
                    dvv = drows[i2, pl.ds(0, L)]
                    ew = sig[0][e] * wcv[OAE + 0, pl.ds(0, L)]
                    for k in range(1, 8):
                        ew = ew + sig[k][e] * wcv[OAE + k, pl.ds(0, L)]
                    ev = jnp.exp(_lk(sv + dvv + ew, 0.2))
                    echunk[i2, pl.ds(0, L)] = ev
                    contrib[i2, pl.ds(0, L)] = ev
                    dvec = drow16[i2, pl.ds(0, L)]
                    for k in range(4):
                        contrib[i2, pl.ds(16 + k * 16, L)] = ev * dvec[k]
                return 0

            lax.fori_loop(0, GRP, grp, 0)
            pltpu.sync_copy(echunk, e1_h.at[pl.ds(base, CH)])
            pltpu.sync_copy(contrib.at[pl.ds(0, 128)], acc_sh.at[dva],
                            add=True)
            pltpu.sync_copy(contrib.at[pl.ds(128, 128)], acc_sh.at[dvb],
                            add=True)
        return 0

    lax.fori_loop(0, 20, chunk_body, 0)
    plsc.subcore_barrier()

    @pl.when(c == 0)
    def _():
        _writeout(acc_sh, acca_h, s)

    @pl.when(c == 1)
    def _():
        _writeout(acc_sh, accb_h, s)


def _sc_gat1(src1, dst1, tsrc, tdst, dtab, rx1, wc):
    return pl.kernel(
        _sc_gat1_body,
        out_type=[jax.ShapeDtypeStruct((E, 16), F32),
                  jax.ShapeDtypeStruct((N, 80), F32),
                  jax.ShapeDtypeStruct((N, 80), F32)],
        mesh=_mesh(),
        compiler_params=_SC_PARAMS,
        scratch_types=[
            pltpu.VMEM((128,), jnp.int32),
            pltpu.VMEM((128,), jnp.int32),
            pltpu.VMEM((128,), jnp.int32),
            pltpu.VMEM((128,), jnp.int32),
            pltpu.VMEM((CH, 16), F32),
            pltpu.VMEM((CH, 16), F32),
            pltpu.VMEM((CH, 16), F32),
            pltpu.VMEM((CH,), F32),
            pltpu.VMEM((CH,), F32),
            pltpu.VMEM((CH, 16), F32),
            pltpu.VMEM((CH, 80), F32),
            pltpu.VMEM((NWC, 16), F32),
            pltpu.VMEM((128, 80), F32),
            pltpu.VMEM_SHARED((NPAD, 80), F32),
        ],
    )(src1, dst1, tsrc, tdst, dtab, rx1, wc)


def _sc_alpha_body(dst_h, e1_h, den_h, al_h,
                   dva, dvb, erows, denrows, achunk):
    c = lax.axis_index("c")
    s = lax.axis_index("s")
    wid = s * NC + c

    def chunk_body(i, _):
        cid = wid + 32 * i

        @pl.when(cid < NCH)
        def _():
            base = cid * CH
            pltpu.sync_copy(dst_h.at[pl.ds(base, 128)], dva)
            pltpu.sync_copy(dst_h.at[pl.ds(base + 128, 128)], dvb)
            pltpu.sync_copy(e1_h.at[pl.ds(base, CH)], erows)
            pltpu.sync_copy(den_h.at[dva], denrows.at[pl.ds(0, 128)])
            pltpu.sync_copy(den_h.at[dvb], denrows.at[pl.ds(128, 128)])

            def row(i2, _r):
                ev = erows[i2, pl.ds(0, L)]
                dn = denrows[i2, pl.ds(0, L)]
                achunk[i2, pl.ds(0, L)] = ev / (dn + 1e-16)
                return 0

            lax.fori_loop(0, CH, row, 0)
            pltpu.sync_copy(achunk, al_h.at[pl.ds(base, CH)])
        return 0

    lax.fori_loop(0, 20, chunk_body, 0)


def _sc_alpha(dst1, e1, den1):
    return pl.kernel(
        _sc_alpha_body,
        out_type=jax.ShapeDtypeStruct((E, 16), F32),
        mesh=_mesh(),
        compiler_params=_SC_PARAMS,
        scratch_types=[
            pltpu.VMEM((128,), jnp.int32),
            pltpu.VMEM((128,), jnp.int32),
            pltpu.VMEM((CH, 16), F32),
            pltpu.VMEM((CH, 16), F32),
            pltpu.VMEM((CH, 16), F32),
        ],
    )(dst1, e1, den1)


def _sc_gat2_body(src_h, dst_h, ts_h, td_h, d1_h, dsta_h, e1_h, den_h,
                  e2_h, acca_h, accb_h, al_h,
                  sva, svb, dva, dvb, srows, drows, d1rows, echunk, contrib,
                  e1rows, denrows, achunk, zbuf, acc_sh):
    c = lax.axis_index("c")
    s = lax.axis_index("s")
    wid = s * NC + c
    _zero_shared(zbuf, acc_sh, s)
    plsc.subcore_barrier()

    def chunk_body(i, _):
        cid = wid + 32 * i

        @pl.when(cid < NCH)
        def _():
            base = cid * CH
            pltpu.sync_copy(src_h.at[pl.ds(base, 128)], sva)
            pltpu.sync_copy(src_h.at[pl.ds(base + 128, 128)], svb)
            pltpu.sync_copy(dst_h.at[pl.ds(base, 128)], dva)
            pltpu.sync_copy(dst_h.at[pl.ds(base + 128, 128)], dvb)
            for j, (sv_, dv_) in enumerate(((sva, dva), (svb, dvb))):
                half = pl.ds(j * 128, 128)
                pltpu.sync_copy(ts_h.at[sv_], srows.at[half])
                pltpu.sync_copy(td_h.at[dv_], drows.at[half])
                pltpu.sync_copy(d1_h.at[sv_], d1rows.at[half])

            def grp(g, _g):
                for e in range(L):
                    i2 = g * L + e
                    sv = srows[i2, pl.ds(0, L)]
                    dvv = drows[i2, pl.ds(0, L)]
                    ev = jnp.exp(_lk(sv + dvv, 0.2))
                    echunk[i2, pl.ds(0, L)] = ev
                    contrib[i2, pl.ds(0, L)] = ev
                    dvec = d1rows[i2, pl.ds(0, L)]
                    for k in range(4):
                        contrib[i2, pl.ds(16 + k * 16, L)] = ev * dvec[k]
                return 0

            lax.fori_loop(0, GRP, grp, 0)
            pltpu.sync_copy(echunk, e2_h.at[pl.ds(base, CH)])
            pltpu.sync_copy(contrib.at[pl.ds(0, 128)], acc_sh.at[dva],
                            add=True)
            pltpu.sync_copy(contrib.at[pl.ds(128, 128)], acc_sh.at[dvb],
                            add=True)
            pltpu.sync_copy(dsta_h.at[pl.ds(base, 128)], sva)
            pltpu.sync_copy(dsta_h.at[pl.ds(base + 128, 128)], svb)
            pltpu.sync_copy(e1_h.at[pl.ds(base, CH)], e1rows)
            pltpu.sync_copy(den_h.at[sva], denrows.at[pl.ds(0, 128)])
            pltpu.sync_copy(den_h.at[svb], denrows.at[pl.ds(128, 128)])

            def row(i2, _r):
                ev = e1rows[i2, pl.ds(0, L)]
                dn = denrows[i2, pl.ds(0, L)]
                achunk[i2, pl.ds(0, L)] = ev / (dn + 1e-16)
                return 0

            lax.fori_loop(0, CH, row, 0)
            pltpu.sync_copy(achunk, al_h.at[pl.ds(base, CH)])
        return 0

    lax.fori_loop(0, 20, chunk_body, 0)
    plsc.subcore_barrier()

    @pl.when(c == 0)
    def _():
        _writeout(acc_sh, acca_h, s)

    @pl.when(c == 1)
    def _():
        _writeout(acc_sh, accb_h, s)


def _sc_gat2(src1, dst1, ts2, td2, d1tab, dst_g1, e1, den1):
    return pl.kernel(
        _sc_gat2_body,
        out_type=[jax.ShapeDtypeStruct((E, 16), F32),
                  jax.ShapeDtypeStruct((N, 80), F32),
                  jax.ShapeDtypeStruct((N, 80), F32),
                  jax.ShapeDtypeStruct((E, 16), F32)],
        mesh=_mesh(),
        compiler_params=_SC_PARAMS,
        scratch_types=[
            pltpu.VMEM((128,), jnp.int32),
            pltpu.VMEM((128,), jnp.int32),
            pltpu.VMEM((128,), jnp.int32),
            pltpu.VMEM((128,), jnp.int32),
            pltpu.VMEM((CH, 16), F32),
            pltpu.VMEM((CH, 16), F32),
            pltpu.VMEM((CH, 16), F32),
            pltpu.VMEM((CH, 16), F32),
            pltpu.VMEM((CH, 80), F32),
            pltpu.VMEM((CH, 16), F32),
            pltpu.VMEM((CH, 16), F32),
            pltpu.VMEM((CH, 16), F32),
            pltpu.VMEM((128, 80), F32),
            pltpu.VMEM_SHARED((NPAD, 80), F32),
        ],
    )(src1, dst1, ts2, td2, d1tab, dst_g1, e1, den1)


def _make_upass_body(kn, koff):
    w = 16 * kn

    def body(src_h, dst_h, e_h, d_h, acca_h, accb_h,
             sva, svb, dva, dvb, erows, drow16, contrib, zbuf, acc_sh):
        c = lax.axis_index("c")
        s = lax.axis_index("s")
        wid = s * NC + c
        _zero_shared(zbuf, acc_sh, s)
        plsc.subcore_barrier()

        def chunk_body(i, _):
            cid = wid + 32 * i

            @pl.when(cid < NCH)
            def _():
                base = cid * CH
                pltpu.sync_copy(src_h.at[pl.ds(base, 128)], sva)
                pltpu.sync_copy(src_h.at[pl.ds(base + 128, 128)], svb)
                pltpu.sync_copy(dst_h.at[pl.ds(base, 128)], dva)
                pltpu.sync_copy(dst_h.at[pl.ds(base + 128, 128)], dvb)
                pltpu.sync_copy(e_h.at[pl.ds(base, CH)], erows)
                pltpu.sync_copy(d_h.at[sva], drow16.at[pl.ds(0, 128)])
                pltpu.sync_copy(d_h.at[svb], drow16.at[pl.ds(128, 128)])

                def grp(g, _g):
                    for e in range(L):
                        i2 = g * L + e
                        ev = erows[i2, pl.ds(0, L)]
                        dvec = drow16[i2, pl.ds(0, L)]
                        for k in range(kn):
                            contrib[i2, pl.ds(k * 16, L)] = ev * dvec[koff + k]
                    return 0

                lax.fori_loop(0, GRP, grp, 0)
                pltpu.sync_copy(contrib.at[pl.ds(0, 128)], acc_sh.at[dva],
                                add=True)
                pltpu.sync_copy(contrib.at[pl.ds(128, 128)], acc_sh.at[dvb],
                                add=True)
            return 0

        lax.fori_loop(0, 20, chunk_body, 0)
        plsc.subcore_barrier()

        @pl.when(c == 0)
        def _():
            _writeout(acc_sh, acca_h, s)

        @pl.when(c == 1)
        def _():
            _writeout(acc_sh, accb_h, s)

    return body


def _sc_upass(src1, dst1, etab, dtab, kn, koff):
    w = 16 * kn
    return pl.kernel(
        _make_upass_body(kn, koff),
        out_type=[jax.ShapeDtypeStruct((N, w), F32),
                  jax.ShapeDtypeStruct((N, w), F32)],
        mesh=_mesh(),
        compiler_params=_SC_PARAMS,
        scratch_types=[
            pltpu.VMEM((128,), jnp.int32),
            pltpu.VMEM((128,), jnp.int32),
            pltpu.VMEM((128,), jnp.int32),
            pltpu.VMEM((128,), jnp.int32),
            pltpu.VMEM((CH, 16), F32),
            pltpu.VMEM((CH, 16), F32),
            pltpu.VMEM((CH, w), F32),
            pltpu.VMEM((128, w), F32),
            pltpu.VMEM_SHARED((NPAD, w), F32),
        ],
    )(src1, dst1, etab, dtab)


def kernel(seq_data, raw_x, edge_index, edge_tf, batch, gene_num, gene_id_vec,
           params):
    p = params
    r2 = lambda a: a.reshape(1, -1)

    w1 = p['c1_w'].reshape(8, H, C)
    as1 = jnp.einsum('khc,hc->kh', w1, p['c1_as'])
    ad1 = jnp.einsum('khc,hc->kh', w1, p['c1_ad'])
    ae1 = jnp.einsum('khc,hc->kh', p['c1_we'].reshape(8, H, C), p['c1_ae'])
    w2 = p['c2_w'].reshape(16, H, C)
    as2 = jnp.einsum('khc,hc->kh', w2, p['c2_as'])
    ad2 = jnp.einsum('khc,hc->kh', w2, p['c2_ad'])
    eye = jnp.eye(H, dtype=F32)
    bd1 = (w1[:, :, None, :] * eye[None, :, :, None]).reshape(128, 256)
    bd2 = (w2[:, :, None, :] * eye[None, :, :, None]).reshape(256, 256)
    wvec = jnp.concatenate([
        p['e_w1'].reshape(-1), p['e_b1'], p['e_g1'], p['e_be1'],
        p['e_w2'].reshape(-1), p['e_b2']])
    wc = jnp.concatenate(
        [jnp.broadcast_to(wvec[:, None], (OAE, L)), ae1], axis=0)

    rx8 = jnp.pad(raw_x, ((0, 0), (0, 7)))
    rx1 = raw_x[:, 0]
    data16, tsrc, tdst = _tc_node(
        rx8, p['n_w1'], r2(p['n_b1']), r2(p['n_g1']), r2(p['n_be1']),
        p['n_w2'], r2(p['n_b2']), r2(p['n_g2']), r2(p['n_be2']),
        p['n_w3'], r2(p['n_b3']), as1, ad1)

    e1, acc1a, acc1b = _sc_gat1(edge_index[0], edge_index[1], tsrc, tdst,
                                data16, rx1, wc)
    u1a, u1b = _sc_upass(edge_index[0], edge_index[1], e1, data16, 4, 4)

    data1, ts2, td2, den1 = _tc_mid(
        acc1a, acc1b, u1a, u1b, bd1, r2(p['c1_b']), p['f1_w'], r2(p['f1_b']),
        as2, ad2)

    e2, acc2a, acc2b, alpha1 = _sc_gat2(edge_tf[0], edge_tf[1], ts2, td2,
                                        data1, edge_index[1], e1, den1)
    u2a, u2b = _sc_upass(edge_tf[0], edge_tf[1], e2, data1, 6, 4)
    u2c, u2d = _sc_upass(edge_tf[0], edge_tf[1], e2, data1, 6, 10)

    dall = _tc_fin(acc2a, acc2b, u2a, u2b, u2c, u2d, data1, bd2,
                   r2(p['c2_b']), p['f2_w'], r2(p['f2_b']))
    gm = _tc_sel(dall)
    gene_out = gm[:, 0]
    cin = gm[:, 1].reshape(1, NG)
    ct = _tc_cell(
        cin, p['ct_w1'], r2(p['ct_b1']), r2(p['ct_g1']), r2(p['ct_be1']),
        p['ct_w2'], r2(p['ct_b2']), r2(p['ct_g2']), r2(p['ct_be2']),
        p['ct_w3'], r2(p['ct_b3']))
    cell_type = ct[0]
    return gene_out, alpha1, cell_type


# speedup vs baseline: 46.6032x; 1.4103x over previous
import functools

import jax
import jax.numpy as jnp
from jax import lax
from jax.experimental import pallas as pl
from jax.experimental.pallas import tpu as pltpu
from jax.experimental.pallas import tpu_sc as plsc

N = 10000
E = 160000
H = 16
C = 16
NG = 2568
F32 = jnp.float32

NC, NS, L = 2, 16, 16
CH = 256
NCH = E // CH
GRP = CH // L
NPAD = 10240

OW1, OB1, OG1, OBE1, OW2, OB2, OAE = 0, 48, 64, 80, 96, 224, 232
NWC = 240

_BLK = 1000
_SC_PARAMS = pltpu.CompilerParams(use_tc_tiling_on_sc=False)


def _ln(x, g, b):
    m = jnp.mean(x, axis=-1, keepdims=True)
    v = jnp.mean((x - m) ** 2, axis=-1, keepdims=True)
    return (x - m) / jnp.sqrt(v + 1e-5) * g + b


def _lk(x, s):
    return jnp.maximum(x, s * x)


def _tc_node_body(rx, w1, b1, g1, be1, w2, b2, g2, be2, w3, b3, as1, ad1,
                  data_o, tsrc_o, tdst_o):
    x = rx[...][:, 0:1]
    h = _lk(_ln(x * w1[...] + b1[...], g1[...], be1[...]), 0.01)
    h = _lk(_ln(jnp.dot(h, w2[...]) + b2[...], g2[...], be2[...]), 0.01)
    d = jnp.dot(h, w3[...]) + b3[...]
    data_o[...] = jnp.concatenate([d, jnp.zeros((x.shape[0], 8), F32)], axis=1)
    tsrc_o[...] = jnp.dot(d, as1[...])
    tdst_o[...] = jnp.dot(d, ad1[...])


def _tc_node(rx8, w1, b1, g1, be1, w2, b2, g2, be2, w3, b3, as1, ad1):
    nb = N // _BLK
    full = lambda a: pl.BlockSpec(a.shape, lambda i: (0,) * a.ndim)
    row = lambda k: pl.BlockSpec((_BLK, k), lambda i: (i, 0))
    args = (w1, b1, g1, be1, w2, b2, g2, be2, w3, b3, as1, ad1)
    return pl.pallas_call(
        _tc_node_body,
        grid=(nb,),
        in_specs=[row(8)] + [full(a) for a in args],
        out_specs=[row(16), row(16), row(16)],
        out_shape=[jax.ShapeDtypeStruct((N, 16), F32),
                   jax.ShapeDtypeStruct((N, 16), F32),
                   jax.ShapeDtypeStruct((N, 16), F32)],
    )(rx8, *args)


def _tc_mid_body(a80, b80, a64, b64, bd1, c1b, f1w, f1b, as2, ad2,
                 d1_o, ts_o, td_o, den_o):
    den = a80[...][:, 0:16] + b80[...][:, 0:16]
    denr = 1.0 / (den + 1e-16)
    u = jnp.concatenate([a80[...][:, 16:80] + b80[...][:, 16:80],
                         a64[...] + b64[...]], axis=1)
    dx = jnp.concatenate([denr] * 8, axis=1)
    t = u * dx
    d1out = jnp.dot(t, bd1[...]) + c1b[...]
    data1 = _lk(jnp.dot(d1out, f1w[...]) + f1b[...], 0.01)
    d1_o[...] = data1
    ts_o[...] = jnp.dot(data1, as2[...])
    td_o[...] = jnp.dot(data1, ad2[...])
    den_o[...] = den


def _tc_mid(acc1a, acc1b, u1a, u1b, bd1, c1b, f1w, f1b, as2, ad2):
    nb = N // _BLK
    full = lambda a: pl.BlockSpec(a.shape, lambda i: (0,) * a.ndim)
    row = lambda k: pl.BlockSpec((_BLK, k), lambda i: (i, 0))
    args = (bd1, c1b, f1w, f1b, as2, ad2)
    return pl.pallas_call(
        _tc_mid_body,
        grid=(nb,),
        in_specs=[row(80), row(80), row(64), row(64)]
        + [full(a) for a in args],
        out_specs=[row(16), row(16), row(16), row(16)],
        out_shape=[jax.ShapeDtypeStruct((N, 16), F32)] * 4,
    )(acc1a, acc1b, u1a, u1b, *args)


def _tc_fin_body(a80, b80, ab1, bb1, ab2, bb2, d1, bd2, c2b, f2w, f2b,
                 dall_o):
    den = a80[...][:, 0:16] + b80[...][:, 0:16]
    denr = 1.0 / (den + 1e-16)
    u = jnp.concatenate([a80[...][:, 16:80] + b80[...][:, 16:80],
                         ab1[...] + bb1[...],
                         ab2[...] + bb2[...]], axis=1)
    dx = jnp.concatenate([denr] * 16, axis=1)
    t = u * dx
    d2out = jnp.dot(t, bd2[...]) + c2b[...]
    data2 = _lk(jnp.dot(d2out, f2w[...]) + f2b[...], 0.01)
    dall_o[...] = d1[...] + data2


def _tc_fin(a80, b80, ab1, bb1, ab2, bb2, data1, bd2, c2b, f2w, f2b):
    nb = N // _BLK
    full = lambda a: pl.BlockSpec(a.shape, lambda i: (0,) * a.ndim)
    row = lambda k: pl.BlockSpec((_BLK, k), lambda i: (i, 0))
    args = (bd2, c2b, f2w, f2b)
    return pl.pallas_call(
        _tc_fin_body,
        grid=(nb,),
        in_specs=[row(80), row(80), row(96), row(96), row(96), row(96),
                  row(16)] + [full(a) for a in args],
        out_specs=[row(16)],
        out_shape=[jax.ShapeDtypeStruct((N, 16), F32)],
    )(a80, b80, ab1, bb1, ab2, bb2, data1, *args)[0]


def _tc_sel_body(dall, gm_o):
    sel = dall[0:NG, :]
    m = jnp.max(sel, axis=1, keepdims=True)
    lse = m[:, 0] + jnp.log(jnp.sum(jnp.exp(sel - m), axis=1))
    gene = lse - sel[:, 0]
    cin = jnp.mean(sel, axis=1)
    z = jnp.zeros((NG, 6), F32)
    gm_o[...] = jnp.concatenate([gene[:, None], cin[:, None], z], axis=1)


def _tc_sel(dall):
    return pl.pallas_call(
        _tc_sel_body,
        out_shape=jax.ShapeDtypeStruct((NG, 8), F32),
    )(dall)


def _tc_cell_body(cin, w1, b1, g1, be1, w2, b2, g2, be2, w3, b3, ct_o):
    c = _lk(_ln(jnp.dot(cin[...], w1[...]) + b1[...], g1[...], be1[...]), 0.01)
    c = _lk(_ln(jnp.dot(c, w2[...]) + b2[...], g2[...], be2[...]), 0.01)
    lg = jnp.dot(c, w3[...]) + b3[...]
    ex = jnp.exp(lg - jnp.max(lg, axis=-1, keepdims=True))
    ct_o[...] = ex / jnp.sum(ex, axis=-1, keepdims=True)


def _tc_cell(cin, *args):
    return pl.pallas_call(
        _tc_cell_body,
        out_shape=jax.ShapeDtypeStruct((1, 19), F32),
    )(cin, *args)


@functools.cache
def _mesh():
    return plsc.VectorSubcoreMesh(core_axis_name="c", subcore_axis_name="s")


def _rsqrt_sc(x):
    i = lax.bitcast_convert_type(x, jnp.int32)
    i = 0x5F3759DF - lax.shift_right_logical(i, 1)
    y = lax.bitcast_convert_type(i, F32)
    for _ in range(3):
        y = y * (1.5 - 0.5 * x * y * y)
    return y


def _zero_shared(zbuf, acc_sh, s):
    def zb(i, _):
        for j in range(zbuf.shape[1] // L):
            zbuf[i, pl.ds(j * L, L)] = jnp.zeros((L,), F32)
        return 0
    lax.fori_loop(0, 128, zb, 0)

    def zc(r, _):
        pltpu.sync_copy(zbuf, acc_sh.at[pl.ds(s * 640 + r * 128, 128)])
        return 0
    lax.fori_loop(0, 5, zc, 0)


def _writeout(acc_sh, out_h, s):
    def wc_(r, _):
        off = s * 640 + r * 128

        @pl.when(off + 128 <= N)
        def _():
            pltpu.sync_copy(acc_sh.at[pl.ds(off, 128)],
                            out_h.at[pl.ds(off, 128)])
        return 0
    lax.fori_loop(0, 5, wc_, 0)

    @pl.when(s == 15)
    def _():
        pltpu.sync_copy(acc_sh.at[pl.ds(9984, 16)], out_h.at[pl.ds(9984, 16)])


def _edge_mlp_group(rs, rd, wcv):
    prod = rs * rd
    hv = []
    for jj in range(16):
        t = (prod * wcv[OW1 + jj, pl.ds(0, L)]
             + rs * wcv[OW1 + 16 + jj, pl.ds(0, L)]
             + rd * wcv[OW1 + 32 + jj, pl.ds(0, L)]
             + wcv[OB1 + jj, pl.ds(0, L)])
        hv.append(t)
    mean = hv[0]
    for t in hv[1:]:
        mean = mean + t
    mean = mean * (1.0 / 16.0)
    dv = [t - mean for t in hv]
    var = dv[0] * dv[0]
    for t in dv[1:]:
        var = var + t * t
    var = var * (1.0 / 16.0)
    r = _rsqrt_sc(var + 1e-5)
    hl = [_lk(dv[jj] * r * wcv[OG1 + jj, pl.ds(0, L)]
              + wcv[OBE1 + jj, pl.ds(0, L)], 0.01)
          for jj in range(16)]
    sig = []
    for jj in range(8):
        t = wcv[OB2 + jj, pl.ds(0, L)]
        for k in range(16):
            t = t + hl[k] * wcv[OW2 + k * 8 + jj, pl.ds(0, L)]
        t = _lk(t, 0.01)
        sig.append(1.0 / (1.0 + jnp.exp(-t)))
    return sig


def _sc_gat1_body(src_h, dst_h, tsrc_h, tdst_h, dtab_h, rx_h, wc_h,
                  e1_h, acca_h, accb_h,
                  sva, svb, dva, dvb, srows, drows, drow16, rxs, rxd,
                  echunk, contrib, wcv, zbuf, acc_sh, sem):
    c = lax.axis_index("c")
    s = lax.axis_index("s")
    wid = s * NC + c
    _zero_shared(zbuf, acc_sh, s)
    pltpu.sync_copy(wc_h, wcv)
    plsc.subcore_barrier()

    def chunk_body(i, _):
        cid = wid + 32 * i

        @pl.when(cid < NCH)
        def _():
            base = cid * CH
            hs = [pltpu.async_copy(src_h.at[pl.ds(base, 128)], sva, sem),
                  pltpu.async_copy(src_h.at[pl.ds(base + 128, 128)], svb, sem),
                  pltpu.async_copy(dst_h.at[pl.ds(base, 128)], dva, sem),
                  pltpu.async_copy(dst_h.at[pl.ds(base + 128, 128)], dvb, sem)]
            for h_ in hs:
                h_.wait()
            hs = []
            for j, (sv_, dv_) in enumerate(((sva, dva), (svb, dvb))):
                half = pl.ds(j * 128, 128)
                hs += [pltpu.async_copy(tsrc_h.at[sv_], srows.at[half], sem),
                       pltpu.async_copy(tdst_h.at[dv_], drows.at[half], sem),
                       pltpu.async_copy(dtab_h.at[sv_], drow16.at[half], sem),
                       pltpu.async_copy(rx_h.at[sv_], rxs.at[half], sem),
                       pltpu.async_copy(rx_h.at[dv_], rxd.at[half], sem)]
            for h_ in hs:
                h_.wait()

            def grp(g, _g):
                rs = rxs[pl.ds(g * L, L)]
                rd = rxd[pl.ds(g * L, L)]
                sig = _edge_mlp_group(rs, rd, wcv)
                for e in range(L):
                    i2 = g * L + e
                    sv = srows[i2, pl.ds(0, L)]
                    dvv = drows[i2, pl.ds(0, L)]
                    ew = sig[0][e] * wcv[OAE + 0, pl.ds(0, L)]
                    for k in range(1, 8):
                        ew = ew + sig[k][e] * wcv[OAE + k, pl.ds(0, L)]
                    ev = jnp.exp(_lk(sv + dvv + ew, 0.2))
                    echunk[i2, pl.ds(0, L)] = ev
                    contrib[i2, pl.ds(0, L)] = ev
                    dvec = drow16[i2, pl.ds(0, L)]
                    for k in range(4):
                        contrib[i2, pl.ds(16 + k * 16, L)] = ev * dvec[k]
                return 0

            lax.fori_loop(0, GRP, grp, 0)
            pltpu.sync_copy(echunk, e1_h.at[pl.ds(base, CH)])
            pltpu.sync_copy(contrib.at[pl.ds(0, 128)], acc_sh.at[dva],
                            add=True)
            pltpu.sync_copy(contrib.at[pl.ds(128, 128)], acc_sh.at[dvb],
                            add=True)
        return 0

    lax.fori_loop(0, 20, chunk_body, 0)
    plsc.subcore_barrier()

    @pl.when(c == 0)
    def _():
        _writeout(acc_sh, acca_h, s)

    @pl.when(c == 1)
    def _():
        _writeout(acc_sh, accb_h, s)


def _sc_gat1(src1, dst1, tsrc, tdst, dtab, rx1, wc):
    return pl.kernel(
        _sc_gat1_body,
        out_type=[jax.ShapeDtypeStruct((E, 16), F32),
                  jax.ShapeDtypeStruct((N, 80), F32),
                  jax.ShapeDtypeStruct((N, 80), F32)],
        mesh=_mesh(),
        compiler_params=_SC_PARAMS,
        scratch_types=[
            pltpu.VMEM((128,), jnp.int32),
            pltpu.VMEM((128,), jnp.int32),
            pltpu.VMEM((128,), jnp.int32),
            pltpu.VMEM((128,), jnp.int32),
            pltpu.VMEM((CH, 16), F32),
            pltpu.VMEM((CH, 16), F32),
            pltpu.VMEM((CH, 16), F32),
            pltpu.VMEM((CH,), F32),
            pltpu.VMEM((CH,), F32),
            pltpu.VMEM((CH, 16), F32),
            pltpu.VMEM((CH, 80), F32),
            pltpu.VMEM((NWC, 16), F32),
            pltpu.VMEM((128, 80), F32),
            pltpu.VMEM_SHARED((NPAD, 80), F32),
            pltpu.SemaphoreType.DMA,
        ],
    )(src1, dst1, tsrc, tdst, dtab, rx1, wc)


def _sc_alpha_body(dst_h, e1_h, den_h, al_h,
                   dva, dvb, erows, denrows, achunk):
    c = lax.axis_index("c")
    s = lax.axis_index("s")
    wid = s * NC + c

    def chunk_body(i, _):
        cid = wid + 32 * i

        @pl.when(cid < NCH)
        def _():
            base = cid * CH
            pltpu.sync_copy(dst_h.at[pl.ds(base, 128)], dva)
            pltpu.sync_copy(dst_h.at[pl.ds(base + 128, 128)], dvb)
            pltpu.sync_copy(e1_h.at[pl.ds(base, CH)], erows)
            pltpu.sync_copy(den_h.at[dva], denrows.at[pl.ds(0, 128)])
            pltpu.sync_copy(den_h.at[dvb], denrows.at[pl.ds(128, 128)])

            def row(i2, _r):
                ev = erows[i2, pl.ds(0, L)]
                dn = denrows[i2, pl.ds(0, L)]
                achunk[i2, pl.ds(0, L)] = ev / (dn + 1e-16)
                return 0

            lax.fori_loop(0, CH, row, 0)
            pltpu.sync_copy(achunk, al_h.at[pl.ds(base, CH)])
        return 0

    lax.fori_loop(0, 20, chunk_body, 0)


def _sc_alpha(dst1, e1, den1):
    return pl.kernel(
        _sc_alpha_body,
        out_type=jax.ShapeDtypeStruct((E, 16), F32),
        mesh=_mesh(),
        compiler_params=_SC_PARAMS,
        scratch_types=[
            pltpu.VMEM((128,), jnp.int32),
            pltpu.VMEM((128,), jnp.int32),
            pltpu.VMEM((CH, 16), F32),
            pltpu.VMEM((CH, 16), F32),
            pltpu.VMEM((CH, 16), F32),
        ],
    )(dst1, e1, den1)


def _sc_gat2_body(src_h, dst_h, ts_h, td_h, d1_h, dsta_h, e1_h, den_h,
                  e2_h, acca_h, accb_h, al_h,
                  sva, svb, dva, dvb, ava, avb, srows, drows, d1rows,
                  echunk, contrib, e1rows, denrows, achunk, zbuf, acc_sh,
                  sem):
    c = lax.axis_index("c")
    s = lax.axis_index("s")
    wid = s * NC + c
    _zero_shared(zbuf, acc_sh, s)
    plsc.subcore_barrier()

    def chunk_body(i, _):
        cid = wid + 32 * i

        @pl.when(cid < NCH)
        def _():
            base = cid * CH
            hs = [pltpu.async_copy(src_h.at[pl.ds(base, 128)], sva, sem),
                  pltpu.async_copy(src_h.at[pl.ds(base + 128, 128)], svb, sem),
                  pltpu.async_copy(dst_h.at[pl.ds(base, 128)], dva, sem),
                  pltpu.async_copy(dst_h.at[pl.ds(base + 128, 128)], dvb, sem),
                  pltpu.async_copy(dsta_h.at[pl.ds(base, 128)], ava, sem),
                  pltpu.async_copy(dsta_h.at[pl.ds(base + 128, 128)], avb,
                                   sem),
                  pltpu.async_copy(e1_h.at[pl.ds(base, CH)], e1rows, sem)]
            for h_ in hs:
                h_.wait()
            hs = [pltpu.async_copy(den_h.at[ava], denrows.at[pl.ds(0, 128)],
                                   sem),
                  pltpu.async_copy(den_h.at[avb], denrows.at[pl.ds(128, 128)],
                                   sem)]
            for j, (sv_, dv_) in enumerate(((sva, dva), (svb, dvb))):
                half = pl.ds(j * 128, 128)
                hs += [pltpu.async_copy(ts_h.at[sv_], srows.at[half], sem),
                       pltpu.async_copy(td_h.at[dv_], drows.at[half], sem),
                       pltpu.async_copy(d1_h.at[sv_], d1rows.at[half], sem)]
            for h_ in hs:
                h_.wait()

            def grp(g, _g):
                for e in range(L):
                    i2 = g * L + e
                    sv = srows[i2, pl.ds(0, L)]
                    dvv = drows[i2, pl.ds(0, L)]
                    ev = jnp.exp(_lk(sv + dvv, 0.2))
                    echunk[i2, pl.ds(0, L)] = ev
                    contrib[i2, pl.ds(0, L)] = ev
                    dvec = d1rows[i2, pl.ds(0, L)]
                    for k in range(4):
                        contrib[i2, pl.ds(16 + k * 16, L)] = ev * dvec[k]
                return 0

            lax.fori_loop(0, GRP, grp, 0)
            pltpu.sync_copy(echunk, e2_h.at[pl.ds(base, CH)])
            pltpu.sync_copy(contrib.at[pl.ds(0, 128)], acc_sh.at[dva],
                            add=True)
            pltpu.sync_copy(contrib.at[pl.ds(128, 128)], acc_sh.at[dvb],
                            add=True)
            def row(i2, _r):
                ev = e1rows[i2, pl.ds(0, L)]
                dn = denrows[i2, pl.ds(0, L)]
                achunk[i2, pl.ds(0, L)] = ev / (dn + 1e-16)
                return 0

            lax.fori_loop(0, CH, row, 0)
            pltpu.sync_copy(achunk, al_h.at[pl.ds(base, CH)])
        return 0

    lax.fori_loop(0, 20, chunk_body, 0)
    plsc.subcore_barrier()

    @pl.when(c == 0)
    def _():
        _writeout(acc_sh, acca_h, s)

    @pl.when(c == 1)
    def _():
        _writeout(acc_sh, accb_h, s)


def _sc_gat2(src1, dst1, ts2, td2, d1tab, dst_g1, e1, den1):
    return pl.kernel(
        _sc_gat2_body,
        out_type=[jax.ShapeDtypeStruct((E, 16), F32),
                  jax.ShapeDtypeStruct((N, 80), F32),
                  jax.ShapeDtypeStruct((N, 80), F32),
                  jax.ShapeDtypeStruct((E, 16), F32)],
        mesh=_mesh(),
        compiler_params=_SC_PARAMS,
        scratch_types=[
            pltpu.VMEM((128,), jnp.int32),
            pltpu.VMEM((128,), jnp.int32),
            pltpu.VMEM((128,), jnp.int32),
            pltpu.VMEM((128,), jnp.int32),
            pltpu.VMEM((128,), jnp.int32),
            pltpu.VMEM((128,), jnp.int32),
            pltpu.VMEM((CH, 16), F32),
            pltpu.VMEM((CH, 16), F32),
            pltpu.VMEM((CH, 16), F32),
            pltpu.VMEM((CH, 16), F32),
            pltpu.VMEM((CH, 80), F32),
            pltpu.VMEM((CH, 16), F32),
            pltpu.VMEM((CH, 16), F32),
            pltpu.VMEM((CH, 16), F32),
            pltpu.VMEM((128, 80), F32),
            pltpu.VMEM_SHARED((NPAD, 80), F32),
            pltpu.SemaphoreType.DMA,
        ],
    )(src1, dst1, ts2, td2, d1tab, dst_g1, e1, den1)


def _make_upass_body(kn, koff):
    w = 16 * kn

    def body(src_h, dst_h, e_h, d_h, acca_h, accb_h,
             sva, svb, dva, dvb, erows, drow16, contrib, zbuf, acc_sh, sem):
        c = lax.axis_index("c")
        s = lax.axis_index("s")
        wid = s * NC + c
        _zero_shared(zbuf, acc_sh, s)
        plsc.subcore_barrier()

        def chunk_body(i, _):
            cid = wid + 32 * i

            @pl.when(cid < NCH)
            def _():
                base = cid * CH
                hs = [pltpu.async_copy(src_h.at[pl.ds(base, 128)], sva, sem),
                      pltpu.async_copy(src_h.at[pl.ds(base + 128, 128)], svb,
                                       sem),
                      pltpu.async_copy(dst_h.at[pl.ds(base, 128)], dva, sem),
                      pltpu.async_copy(dst_h.at[pl.ds(base + 128, 128)], dvb,
                                       sem),
                      pltpu.async_copy(e_h.at[pl.ds(base, CH)], erows, sem)]
                for h_ in hs:
                    h_.wait()
                hs = [pltpu.async_copy(d_h.at[sva], drow16.at[pl.ds(0, 128)],
                                       sem),
                      pltpu.async_copy(d_h.at[svb],
                                       drow16.at[pl.ds(128, 128)], sem)]
                for h_ in hs:
                    h_.wait()

                def grp(g, _g):
                    for e in range(L):
                        i2 = g * L + e
                        ev = erows[i2, pl.ds(0, L)]
                        dvec = drow16[i2, pl.ds(0, L)]
                        for k in range(kn):
                            contrib[i2, pl.ds(k * 16, L)] = ev * dvec[koff + k]
                    return 0

                lax.fori_loop(0, GRP, grp, 0)
                pltpu.sync_copy(contrib.at[pl.ds(0, 128)], acc_sh.at[dva],
                                add=True)
                pltpu.sync_copy(contrib.at[pl.ds(128, 128)], acc_sh.at[dvb],
                                add=True)
            return 0

        lax.fori_loop(0, 20, chunk_body, 0)
        plsc.subcore_barrier()

        @pl.when(c == 0)
        def _():
            _writeout(acc_sh, acca_h, s)

        @pl.when(c == 1)
        def _():
            _writeout(acc_sh, accb_h, s)

    return body


def _sc_upass(src1, dst1, etab, dtab, kn, koff):
    w = 16 * kn
    return pl.kernel(
        _make_upass_body(kn, koff),
        out_type=[jax.ShapeDtypeStruct((N, w), F32),
                  jax.ShapeDtypeStruct((N, w), F32)],
        mesh=_mesh(),
        compiler_params=_SC_PARAMS,
        scratch_types=[
            pltpu.VMEM((128,), jnp.int32),
            pltpu.VMEM((128,), jnp.int32),
            pltpu.VMEM((128,), jnp.int32),
            pltpu.VMEM((128,), jnp.int32),
            pltpu.VMEM((CH, 16), F32),
            pltpu.VMEM((CH, 16), F32),
            pltpu.VMEM((CH, w), F32),
            pltpu.VMEM((128, w), F32),
            pltpu.VMEM_SHARED((NPAD, w), F32),
            pltpu.SemaphoreType.DMA,
        ],
    )(src1, dst1, etab, dtab)


def kernel(seq_data, raw_x, edge_index, edge_tf, batch, gene_num, gene_id_vec,
           params):
    p = params
    r2 = lambda a: a.reshape(1, -1)

    w1 = p['c1_w'].reshape(8, H, C)
    as1 = jnp.einsum('khc,hc->kh', w1, p['c1_as'])
    ad1 = jnp.einsum('khc,hc->kh', w1, p['c1_ad'])
    ae1 = jnp.einsum('khc,hc->kh', p['c1_we'].reshape(8, H, C), p['c1_ae'])
    w2 = p['c2_w'].reshape(16, H, C)
    as2 = jnp.einsum('khc,hc->kh', w2, p['c2_as'])
    ad2 = jnp.einsum('khc,hc->kh', w2, p['c2_ad'])
    eye = jnp.eye(H, dtype=F32)
    bd1 = (w1[:, :, None, :] * eye[None, :, :, None]).reshape(128, 256)
    bd2 = (w2[:, :, None, :] * eye[None, :, :, None]).reshape(256, 256)
    wvec = jnp.concatenate([
        p['e_w1'].reshape(-1), p['e_b1'], p['e_g1'], p['e_be1'],
        p['e_w2'].reshape(-1), p['e_b2']])
    wc = jnp.concatenate(
        [jnp.broadcast_to(wvec[:, None], (OAE, L)), ae1], axis=0)

    rx8 = jnp.pad(raw_x, ((0, 0), (0, 7)))
    rx1 = raw_x[:, 0]
    data16, tsrc, tdst = _tc_node(
        rx8, p['n_w1'], r2(p['n_b1']), r2(p['n_g1']), r2(p['n_be1']),
        p['n_w2'], r2(p['n_b2']), r2(p['n_g2']), r2(p['n_be2']),
        p['n_w3'], r2(p['n_b3']), as1, ad1)

    e1, acc1a, acc1b = _sc_gat1(edge_index[0], edge_index[1], tsrc, tdst,
                                data16, rx1, wc)
    u1a, u1b = _sc_upass(edge_index[0], edge_index[1], e1, data16, 4, 4)

    data1, ts2, td2, den1 = _tc_mid(
        acc1a, acc1b, u1a, u1b, bd1, r2(p['c1_b']), p['f1_w'], r2(p['f1_b']),
        as2, ad2)

    e2, acc2a, acc2b, alpha1 = _sc_gat2(edge_tf[0], edge_tf[1], ts2, td2,
                                        data1, edge_index[1], e1, den1)
    u2a, u2b = _sc_upass(edge_tf[0], edge_tf[1], e2, data1, 6, 4)
    u2c, u2d = _sc_upass(edge_tf[0], edge_tf[1], e2, data1, 6, 10)

    dall = _tc_fin(acc2a, acc2b, u2a, u2b, u2c, u2d, data1, bd2,
                   r2(p['c2_b']), p['f2_w'], r2(p['f2_b']))
    gm = _tc_sel(dall)
    gene_out = gm[:, 0]
    cin = gm[:, 1].reshape(1, NG)
    ct = _tc_cell(
        cin, p['ct_w1'], r2(p['ct_b1']), r2(p['ct_g1']), r2(p['ct_be1']),
        p['ct_w2'], r2(p['ct_b2']), r2(p['ct_g2']), r2(p['ct_be2']),
        p['ct_w3'], r2(p['ct_b3']))
    cell_type = ct[0]
    return gene_out, alpha1, cell_type
